# Initial kernel scaffold; baseline (speedup 1.0000x reference)
import jax
import jax.numpy as jnp
from jax.experimental import pallas as pl


def kernel(s, v, p, e, batch, edge_index_global, W_shared, b_shared, W_bond, b_bond, W_b0, b_b0, W_b1, b_b1, W_coords, W_atoms, b_atoms):
    raise NotImplementedError("write your pallas kernel here")


# speedup vs baseline: 12.0000x; 12.0000x over previous
import functools

import jax
import jax.numpy as jnp
from jax import lax
from jax.experimental import pallas as pl
from jax.experimental.pallas import tpu as pltpu
from jax.experimental.pallas import tpu_sc as plsc

N = 1024
E = 65536
SDIM = 256
VDIM = 64
EDIM = 16
NAF = 16
NBT = 5
NG = 64

PAD_ROWS = 256
EBLK = 512
NBLK = E // EBLK
CH = 2048
SUB = 128

_F32 = jnp.float32
_HI = lax.Precision.HIGHEST


def _stage_a_body(s_ref, vT_ref, pT_ref, batch_ref, Wsh_ref, bsh_ref,
                  W3T_ref, Wat_ref, bat_ref, W0_ref, bb0_ref, Wbond_ref,
                  bbond_ref, t_ref, atoms_ref, cpT_ref, Wc2_ref, bvec_ref):
    s2 = jax.nn.silu(
        jnp.dot(s_ref[...], Wsh_ref[...], preferred_element_type=_F32,
                precision=_HI) + bsh_ref[...])
    t_ref[...] = jnp.dot(s2, W0_ref[...], preferred_element_type=_F32,
                         precision=_HI)
    atoms_ref[...] = jnp.dot(s2, Wat_ref[...], preferred_element_type=_F32,
                             precision=_HI) + bat_ref[...]
    cp0 = jnp.dot(W3T_ref[...], vT_ref[...], preferred_element_type=_F32,
                  precision=_HI) + pT_ref[...]
    bI = (lax.broadcasted_iota(jnp.int32, (NG, N), 0)
          == batch_ref[...]).astype(_F32)
    counts_row = lax.dot_general(jnp.ones((1, N), _F32), bI,
                                 (((1,), (1,)), ((), ())),
                                 preferred_element_type=_F32)
    segT = lax.dot_general(cp0, bI, (((1,), (1,)), ((), ())),
                           preferred_element_type=_F32, precision=_HI)
    meanT = segT / jnp.maximum(counts_row, 1.0)
    cpT_ref[...] = cp0 - jnp.dot(meanT, bI, preferred_element_type=_F32,
                                 precision=_HI)
    Wc2_ref[...] = 0.5 * jnp.dot(Wbond_ref[...], W0_ref[...],
                                 preferred_element_type=_F32, precision=_HI)
    bvec_ref[...] = jnp.dot(bbond_ref[...], W0_ref[...],
                            preferred_element_type=_F32,
                            precision=_HI) + bb0_ref[...]


def _stage_a(s, vT, pT, batch2, W_shared, bsh, W3T, W_atoms, bat, W0, bb0,
             W_bond, bbond):
    return pl.pallas_call(
        _stage_a_body,
        out_shape=[
            jax.ShapeDtypeStruct((N, SDIM), _F32),
            jax.ShapeDtypeStruct((N, NAF), _F32),
            jax.ShapeDtypeStruct((3, N), _F32),
            jax.ShapeDtypeStruct((EDIM, SDIM), _F32),
            jax.ShapeDtypeStruct((1, SDIM), _F32),
        ],
    )(s, vT, pT, batch2, W_shared, bsh, W3T, W_atoms, bat, W0, bb0, W_bond,
      bbond)


def _make_sc_table(nc, ns):
    nw = nc * ns
    spw = (N * N) // nw
    mesh = plsc.VectorSubcoreMesh(core_axis_name="c", subcore_axis_name="s",
                                  num_cores=nc, num_subcores=ns)

    @functools.partial(
        pl.kernel,
        out_type=jax.ShapeDtypeStruct((N * N,), jnp.int32),
        mesh=mesh,
        compiler_params=pltpu.CompilerParams(needs_layout_passes=False),
        scratch_types=[
            pltpu.VMEM((CH,), jnp.int32),
            pltpu.VMEM((CH,), jnp.int32),
            pltpu.VMEM((spw,), jnp.int32),
        ],
    )
    def sc_table(j_hbm, i_hbm, T_hbm, jv, iv, Lb):
        wid = lax.axis_index("s") * nc + lax.axis_index("c")
        lo = wid * spw

        def init(q, carry):
            Lb[pl.ds(q * 16, 16)] = jnp.full((16,), -1, jnp.int32)
            return carry
        lax.fori_loop(0, spw // 16, init, 0)

        def chunk(c, carry):
            pltpu.sync_copy(j_hbm.at[pl.ds(c * CH, CH)], jv)
            pltpu.sync_copy(i_hbm.at[pl.ds(c * CH, CH)], iv)

            def scan(q, carry2):
                j16 = jv[pl.ds(q * 16, 16)]
                i16 = iv[pl.ds(q * 16, 16)]
                key = j16 * N + i16
                loc = key - lo
                m = (loc >= 0) & (loc < spw)
                locs = jnp.where(m, loc, 0)
                kvec = lax.iota(jnp.int32, 16) + (c * CH + q * 16)
                plsc.store_scatter(Lb, [locs], kvec, mask=m)
                return carry2
            lax.fori_loop(0, CH // 16, scan, 0)
            return carry
        lax.fori_loop(0, E // CH, chunk, 0)
        pltpu.sync_copy(Lb, T_hbm.at[pl.ds(lo, spw)])

    return sc_table


def _make_sc_gather(nc, ns):
    nw = nc * ns
    epw = E // nw
    nsub = epw // SUB
    mesh = plsc.VectorSubcoreMesh(core_axis_name="c", subcore_axis_name="s",
                                  num_cores=nc, num_subcores=ns)

    @functools.partial(
        pl.kernel,
        out_type=[
            jax.ShapeDtypeStruct((E * EDIM,), _F32),
            jax.ShapeDtypeStruct((E * EDIM,), _F32),
            jax.ShapeDtypeStruct((E,), _F32),
        ],
        mesh=mesh,
        compiler_params=pltpu.CompilerParams(needs_layout_passes=False),
        scratch_types=[
            pltpu.VMEM((epw,), jnp.int32),
            pltpu.VMEM((epw,), jnp.int32),
            pltpu.VMEM((epw,), jnp.int32),
            pltpu.VMEM((epw,), jnp.int32),
            pltpu.VMEM((epw,), jnp.int32),
            pltpu.VMEM((epw,), jnp.int32),
            pltpu.VMEM((epw * EDIM,), _F32),
            pltpu.VMEM((epw * EDIM,), _F32),
            pltpu.VMEM((SUB, 128), _F32),
            pltpu.VMEM((SUB, 128), _F32),
            pltpu.VMEM((epw,), _F32),
            pltpu.VMEM((N,), _F32),
            pltpu.VMEM((N,), _F32),
            pltpu.VMEM((N,), _F32),
            pltpu.SemaphoreType.DMA,
            pltpu.SemaphoreType.DMA,
        ],
    )
    def sc_gather(T_hbm, j_hbm, i_hbm, epad_hbm, cpT_hbm,
                  ef_hbm, er_hbm, d_hbm,
                  jv, iv, keyb, rkeyb, fidb, ridb, efb, erb,
                  slabA, slabB, db, cx, cy, cz, semA, semB):
        wid = lax.axis_index("s") * nc + lax.axis_index("c")
        base = wid * epw
        pltpu.sync_copy(j_hbm.at[pl.ds(base, epw)], jv)
        pltpu.sync_copy(i_hbm.at[pl.ds(base, epw)], iv)
        pltpu.sync_copy(cpT_hbm.at[pl.ds(0, N)], cx)
        pltpu.sync_copy(cpT_hbm.at[pl.ds(N, N)], cy)
        pltpu.sync_copy(cpT_hbm.at[pl.ds(2 * N, N)], cz)

        def phase1(q, carry):
            sl = pl.ds(q * 16, 16)
            j16 = jv[sl]
            i16 = iv[sl]
            keyb[sl] = j16 * N + i16
            rkeyb[sl] = i16 * N + j16
            xi = plsc.load_gather(cx, [i16])
            xj = plsc.load_gather(cx, [j16])
            yi = plsc.load_gather(cy, [i16])
            yj = plsc.load_gather(cy, [j16])
            zi = plsc.load_gather(cz, [i16])
            zj = plsc.load_gather(cz, [j16])
            dx = xi - xj
            dy = yi - yj
            dz = zi - zj
            db[sl] = dx * dx + dy * dy + dz * dz
            return carry
        lax.fori_loop(0, epw // 16, phase1, 0)

        cps = []
        for b in range(nsub):
            sl = pl.ds(b * SUB, SUB)
            cps.append(pltpu.async_copy(T_hbm.at[keyb.at[sl]], fidb.at[sl],
                                        semA))
        for cp in cps:
            cp.wait()
        cps = []
        for b in range(nsub):
            sl = pl.ds(b * SUB, SUB)
            cps.append(pltpu.async_copy(T_hbm.at[rkeyb.at[sl]], ridb.at[sl],
                                        semA))
        for cp in cps:
            cp.wait()

        def phase2(q, carry):
            sl = pl.ds(q * 16, 16)
            r16 = ridb[sl]
            pad = E + ((lax.iota(jnp.int32, 16) + (q * 16 + base))
                       & (PAD_ROWS - 1))
            ridb[sl] = jnp.where(r16 < 0, pad, r16)
            return carry
        lax.fori_loop(0, epw // 16, phase2, 0)

        def run_dir(idx_ref, out_ref):
            slabs = (slabA, slabB)
            sems = (semA, semB)
            cps = [None, None]
            cps[0] = pltpu.async_copy(epad_hbm.at[idx_ref.at[pl.ds(0, SUB)]],
                                      slabs[0], sems[0])
            for b in range(nsub):
                if b + 1 < nsub:
                    cps[(b + 1) % 2] = pltpu.async_copy(
                        epad_hbm.at[idx_ref.at[pl.ds((b + 1) * SUB, SUB)]],
                        slabs[(b + 1) % 2], sems[(b + 1) % 2])
                cps[b % 2].wait()
                slab = slabs[b % 2]

                def extract(r, carry):
                    row = slab[r, pl.ds(0, 16)]
                    out_ref[pl.ds((b * SUB + r) * 16, 16)] = row
                    return carry
                lax.fori_loop(0, SUB, extract, 0)

        run_dir(fidb, efb)
        run_dir(ridb, erb)

        pltpu.sync_copy(efb, ef_hbm.at[pl.ds(base * EDIM, epw * EDIM)])
        pltpu.sync_copy(erb, er_hbm.at[pl.ds(base * EDIM, epw * EDIM)])
        pltpu.sync_copy(db, d_hbm.at[pl.ds(base, epw)])

    return sc_gather


def _stage_c_body(iR, jR, dR, efR, erR, tR, Wc2R, wdR, bvecR, Wb1R, bb1R, oR):
    icol = iR[0]
    jcol = jR[0]
    ni = lax.broadcasted_iota(jnp.int32, (EBLK, N), 1)
    oh = (ni == icol).astype(_F32) + (ni == jcol).astype(_F32)
    h = jnp.dot(oh, tR[...], preferred_element_type=_F32, precision=_HI)
    e2 = efR[...] + erR[...]
    h = h + jnp.dot(e2, Wc2R[...], preferred_element_type=_F32,
                    precision=_HI)
    h = h + dR[0] * wdR[...] + bvecR[...]
    sil = h * jax.nn.sigmoid(h)
    oR[...] = jnp.dot(sil, Wb1R[...], preferred_element_type=_F32,
                      precision=_HI) + bb1R[...]


def _stage_c(iR, jR, dR, ef, er, t, Wc2, wd, bvec, W_b1, bb1):
    return pl.pallas_call(
        _stage_c_body,
        grid=(NBLK,),
        in_specs=[
            pl.BlockSpec((1, EBLK, 1), lambda g: (g, 0, 0)),
            pl.BlockSpec((1, EBLK, 1), lambda g: (g, 0, 0)),
            pl.BlockSpec((1, EBLK, 1), lambda g: (g, 0, 0)),
            pl.BlockSpec((EBLK, EDIM), lambda g: (g, 0)),
            pl.BlockSpec((EBLK, EDIM), lambda g: (g, 0)),
            pl.BlockSpec((N, SDIM), lambda g: (0, 0)),
            pl.BlockSpec((EDIM, SDIM), lambda g: (0, 0)),
            pl.BlockSpec((1, SDIM), lambda g: (0, 0)),
            pl.BlockSpec((1, SDIM), lambda g: (0, 0)),
            pl.BlockSpec((SDIM, NBT), lambda g: (0, 0)),
            pl.BlockSpec((1, NBT), lambda g: (0, 0)),
        ],
        out_specs=pl.BlockSpec((EBLK, NBT), lambda g: (g, 0)),
        out_shape=jax.ShapeDtypeStruct((E, NBT), _F32),
    )(iR, jR, dR, ef, er, t, Wc2, wd, bvec, W_b1, bb1)


def kernel(s, v, p, e, batch, edge_index_global,
           W_shared, b_shared, W_bond, b_bond,
           W_b0, b_b0, W_b1, b_b1,
           W_coords, W_atoms, b_atoms):
    try:
        info = plsc.get_sparse_core_info()
        nc, ns = info.num_cores, info.num_subcores
    except Exception:
        nc, ns = 2, 16

    j_ = edge_index_global[0].astype(jnp.int32)
    i_ = edge_index_global[1].astype(jnp.int32)
    batch2 = batch.astype(jnp.int32).reshape(1, N)
    vT = v.reshape(N, 3 * VDIM).T
    pT = p.T
    z = jnp.zeros((VDIM, 1), _F32)
    W3 = jnp.concatenate([
        jnp.concatenate([W_coords, z, z], axis=0),
        jnp.concatenate([z, W_coords, z], axis=0),
        jnp.concatenate([z, z, W_coords], axis=0),
    ], axis=1)
    W3T = W3.T
    W0 = W_b0[:SDIM]
    wd = W_b0[SDIM:SDIM + 1]
    bsh = b_shared.reshape(1, SDIM)
    bat = b_atoms.reshape(1, NAF)
    bb0 = b_b0.reshape(1, SDIM)
    bbond = b_bond.reshape(1, SDIM)
    bb1 = b_b1.reshape(1, NBT)
    epad = jnp.pad(e, ((0, PAD_ROWS), (0, 128 - EDIM)))

    t, atoms_pred, cpT, Wc2, bvec = _stage_a(
        s, vT, pT, batch2, W_shared, bsh, W3T, W_atoms, bat, W0, bb0,
        W_bond, bbond)

    T = _make_sc_table(nc, ns)(j_, i_)
    cpflat = cpT.reshape(3 * N)
    ef, er, dsq = _make_sc_gather(nc, ns)(T, j_, i_, epad, cpflat)

    iR = i_.reshape(NBLK, EBLK, 1)
    jR = j_.reshape(NBLK, EBLK, 1)
    dR = dsq.reshape(NBLK, EBLK, 1)
    ef2 = ef.reshape(E, EDIM)
    er2 = er.reshape(E, EDIM)
    bonds_pred = _stage_c(iR, jR, dR, ef2, er2, t, Wc2, wd, bvec, W_b1, bb1)

    coords_pred = cpT.T
    return coords_pred, atoms_pred, bonds_pred


# speedup vs baseline: 15.7831x; 1.3153x over previous
import functools

import jax
import jax.numpy as jnp
from jax import lax
from jax.experimental import pallas as pl
from jax.experimental.pallas import tpu as pltpu
from jax.experimental.pallas import tpu_sc as plsc

N = 1024
E = 65536
SDIM = 256
VDIM = 64
EDIM = 16
NAF = 16
NBT = 5
NG = 64

PAD_ROWS = 256
EBLK = 512
NBLK = E // EBLK
CH = 2048
SUB = 128

_F32 = jnp.float32
_HI = lax.Precision.HIGHEST


def _stage_a_body(s_ref, vT_ref, pT_ref, batch_ref, Wsh_ref, bsh_ref,
                  W3T_ref, Wat_ref, bat_ref, W0_ref, bb0_ref, Wbond_ref,
                  bbond_ref, t_ref, atoms_ref, cpT_ref, Wc2_ref, bvec_ref):
    s2 = jax.nn.silu(
        jnp.dot(s_ref[...], Wsh_ref[...], preferred_element_type=_F32,
                precision=_HI) + bsh_ref[...])
    t_ref[...] = jnp.dot(s2, W0_ref[...], preferred_element_type=_F32,
                         precision=_HI)
    atoms_ref[...] = jnp.dot(s2, Wat_ref[...], preferred_element_type=_F32,
                             precision=_HI) + bat_ref[...]
    cp0 = jnp.dot(W3T_ref[...], vT_ref[...], preferred_element_type=_F32,
                  precision=_HI) + pT_ref[...]
    bI = (lax.broadcasted_iota(jnp.int32, (NG, N), 0)
          == batch_ref[...]).astype(_F32)
    counts_row = lax.dot_general(jnp.ones((1, N), _F32), bI,
                                 (((1,), (1,)), ((), ())),
                                 preferred_element_type=_F32)
    segT = lax.dot_general(cp0, bI, (((1,), (1,)), ((), ())),
                           preferred_element_type=_F32, precision=_HI)
    meanT = segT / jnp.maximum(counts_row, 1.0)
    cpT_ref[...] = cp0 - jnp.dot(meanT, bI, preferred_element_type=_F32,
                                 precision=_HI)
    Wc2_ref[...] = 0.5 * jnp.dot(Wbond_ref[...], W0_ref[...],
                                 preferred_element_type=_F32, precision=_HI)
    bvec_ref[...] = jnp.dot(bbond_ref[...], W0_ref[...],
                            preferred_element_type=_F32,
                            precision=_HI) + bb0_ref[...]


def _stage_a(s, vT, pT, batch2, W_shared, bsh, W3T, W_atoms, bat, W0, bb0,
             W_bond, bbond):
    return pl.pallas_call(
        _stage_a_body,
        out_shape=[
            jax.ShapeDtypeStruct((N, SDIM), _F32),
            jax.ShapeDtypeStruct((N, NAF), _F32),
            jax.ShapeDtypeStruct((3, N), _F32),
            jax.ShapeDtypeStruct((EDIM, SDIM), _F32),
            jax.ShapeDtypeStruct((1, SDIM), _F32),
        ],
    )(s, vT, pT, batch2, W_shared, bsh, W3T, W_atoms, bat, W0, bb0, W_bond,
      bbond)


def _make_sc_table(nc, ns):
    nw = nc * ns
    spw = (N * N) // nw
    mesh = plsc.VectorSubcoreMesh(core_axis_name="c", subcore_axis_name="s",
                                  num_cores=nc, num_subcores=ns)

    @functools.partial(
        pl.kernel,
        out_type=jax.ShapeDtypeStruct((N * N,), jnp.int32),
        mesh=mesh,
        compiler_params=pltpu.CompilerParams(needs_layout_passes=False),
        scratch_types=[
            pltpu.VMEM((CH,), jnp.int32),
            pltpu.VMEM((CH,), jnp.int32),
            pltpu.VMEM((spw,), jnp.int32),
        ],
    )
    def sc_table(j_hbm, i_hbm, T_hbm, jv, iv, Lb):
        wid = lax.axis_index("s") * nc + lax.axis_index("c")
        lo = wid * spw

        def init(q, carry):
            Lb[pl.ds(q * 16, 16)] = jnp.full((16,), -1, jnp.int32)
            return carry
        lax.fori_loop(0, spw // 16, init, 0)

        def chunk(c, carry):
            pltpu.sync_copy(j_hbm.at[pl.ds(c * CH, CH)], jv)
            pltpu.sync_copy(i_hbm.at[pl.ds(c * CH, CH)], iv)

            def scan(q, carry2):
                j16 = jv[pl.ds(q * 16, 16)]
                i16 = iv[pl.ds(q * 16, 16)]
                key = j16 * N + i16
                loc = key - lo
                m = (loc >= 0) & (loc < spw)
                kvec = lax.iota(jnp.int32, 16) + (c * CH + q * 16)
                plsc.store_scatter(Lb, [loc], kvec, mask=m)
                return carry2
            lax.fori_loop(0, CH // 16, scan, 0)
            return carry
        lax.fori_loop(0, E // CH, chunk, 0)
        pltpu.sync_copy(Lb, T_hbm.at[pl.ds(lo, spw)])

    return sc_table


def _make_sc_gather(nc, ns):
    nw = nc * ns
    epw = E // nw
    nsub = epw // SUB
    mesh = plsc.VectorSubcoreMesh(core_axis_name="c", subcore_axis_name="s",
                                  num_cores=nc, num_subcores=ns)

    @functools.partial(
        pl.kernel,
        out_type=[
            jax.ShapeDtypeStruct((E * EDIM,), _F32),
            jax.ShapeDtypeStruct((E * EDIM,), _F32),
            jax.ShapeDtypeStruct((E,), _F32),
        ],
        mesh=mesh,
        compiler_params=pltpu.CompilerParams(needs_layout_passes=False),
        scratch_types=[
            pltpu.VMEM((epw,), jnp.int32),
            pltpu.VMEM((epw,), jnp.int32),
            pltpu.VMEM((epw,), jnp.int32),
            pltpu.VMEM((epw,), jnp.int32),
            pltpu.VMEM((epw,), jnp.int32),
            pltpu.VMEM((epw,), jnp.int32),
            pltpu.VMEM((epw * EDIM,), _F32),
            pltpu.VMEM((epw * EDIM,), _F32),
            pltpu.VMEM((SUB, 128), _F32),
            pltpu.VMEM((SUB, 128), _F32),
            pltpu.VMEM((epw,), _F32),
            pltpu.VMEM((N,), _F32),
            pltpu.VMEM((N,), _F32),
            pltpu.VMEM((N,), _F32),
            pltpu.SemaphoreType.DMA,
            pltpu.SemaphoreType.DMA,
        ],
    )
    def sc_gather(T_hbm, j_hbm, i_hbm, epad_hbm, cpT_hbm,
                  ef_hbm, er_hbm, d_hbm,
                  jv, iv, keyb, rkeyb, fidb, ridb, efb, erb,
                  slabA, slabB, db, cx, cy, cz, semA, semB):
        wid = lax.axis_index("s") * nc + lax.axis_index("c")
        base = wid * epw
        pltpu.sync_copy(j_hbm.at[pl.ds(base, epw)], jv)
        pltpu.sync_copy(i_hbm.at[pl.ds(base, epw)], iv)
        pltpu.sync_copy(cpT_hbm.at[pl.ds(0, N)], cx)
        pltpu.sync_copy(cpT_hbm.at[pl.ds(N, N)], cy)
        pltpu.sync_copy(cpT_hbm.at[pl.ds(2 * N, N)], cz)

        def phase1(q, carry):
            sl = pl.ds(q * 16, 16)
            j16 = jv[sl]
            i16 = iv[sl]
            keyb[sl] = j16 * N + i16
            rkeyb[sl] = i16 * N + j16
            xi = plsc.load_gather(cx, [i16])
            xj = plsc.load_gather(cx, [j16])
            yi = plsc.load_gather(cy, [i16])
            yj = plsc.load_gather(cy, [j16])
            zi = plsc.load_gather(cz, [i16])
            zj = plsc.load_gather(cz, [j16])
            dx = xi - xj
            dy = yi - yj
            dz = zi - zj
            db[sl] = dx * dx + dy * dy + dz * dz
            return carry
        lax.fori_loop(0, epw // 16, phase1, 0)

        cps = []
        for b in range(nsub):
            sl = pl.ds(b * SUB, SUB)
            cps.append(pltpu.async_copy(T_hbm.at[keyb.at[sl]], fidb.at[sl],
                                        semA))
        for cp in cps:
            cp.wait()
        cps = []
        for b in range(nsub):
            sl = pl.ds(b * SUB, SUB)
            cps.append(pltpu.async_copy(T_hbm.at[rkeyb.at[sl]], ridb.at[sl],
                                        semA))
        for cp in cps:
            cp.wait()

        def phase2(q, carry):
            sl = pl.ds(q * 16, 16)
            r16 = ridb[sl]
            pad = E + ((lax.iota(jnp.int32, 16) + (q * 16 + base))
                       & (PAD_ROWS - 1))
            ridb[sl] = jnp.where(r16 < 0, pad, r16)
            return carry
        lax.fori_loop(0, epw // 16, phase2, 0)

        def run_dir(idx_ref, out_ref):
            slabs = (slabA, slabB)
            sems = (semA, semB)
            cps = [None, None]
            cps[0] = pltpu.async_copy(epad_hbm.at[idx_ref.at[pl.ds(0, SUB)]],
                                      slabs[0], sems[0])
            for b in range(nsub):
                if b + 1 < nsub:
                    cps[(b + 1) % 2] = pltpu.async_copy(
                        epad_hbm.at[idx_ref.at[pl.ds((b + 1) * SUB, SUB)]],
                        slabs[(b + 1) % 2], sems[(b + 1) % 2])
                cps[b % 2].wait()
                slab = slabs[b % 2]

                def extract(r, carry):
                    out_ref[pl.ds((b * SUB + r) * 16, 16)] = slab[r, pl.ds(0, 16)]
                    return carry
                lax.fori_loop(0, SUB, extract, 0)

        run_dir(fidb, efb)
        run_dir(ridb, erb)

        pltpu.sync_copy(efb, ef_hbm.at[pl.ds(base * EDIM, epw * EDIM)])
        pltpu.sync_copy(erb, er_hbm.at[pl.ds(base * EDIM, epw * EDIM)])
        pltpu.sync_copy(db, d_hbm.at[pl.ds(base, epw)])

    return sc_gather


def _stage_c_body(iR, jR, dR, efR, erR, tR, Wc2R, wdR, bvecR, Wb1R, bb1R, oR):
    icol = iR[0]
    jcol = jR[0]
    ni = lax.broadcasted_iota(jnp.int32, (EBLK, N), 1)
    oh = (ni == icol).astype(_F32) + (ni == jcol).astype(_F32)
    h = jnp.dot(oh, tR[...], preferred_element_type=_F32)
    e2 = efR[...] + erR[...]
    h = h + jnp.dot(e2, Wc2R[...], preferred_element_type=_F32,
                    precision=_HI)
    h = h + dR[0] * wdR[...] + bvecR[...]
    sil = h * jax.nn.sigmoid(h)
    oR[...] = jnp.dot(sil, Wb1R[...], preferred_element_type=_F32,
                      precision=_HI) + bb1R[...]


def _stage_c(iR, jR, dR, ef, er, t, Wc2, wd, bvec, W_b1, bb1):
    return pl.pallas_call(
        _stage_c_body,
        grid=(NBLK,),
        in_specs=[
            pl.BlockSpec((1, EBLK, 1), lambda g: (g, 0, 0)),
            pl.BlockSpec((1, EBLK, 1), lambda g: (g, 0, 0)),
            pl.BlockSpec((1, EBLK, 1), lambda g: (g, 0, 0)),
            pl.BlockSpec((EBLK, EDIM), lambda g: (g, 0)),
            pl.BlockSpec((EBLK, EDIM), lambda g: (g, 0)),
            pl.BlockSpec((N, SDIM), lambda g: (0, 0)),
            pl.BlockSpec((EDIM, SDIM), lambda g: (0, 0)),
            pl.BlockSpec((1, SDIM), lambda g: (0, 0)),
            pl.BlockSpec((1, SDIM), lambda g: (0, 0)),
            pl.BlockSpec((SDIM, NBT), lambda g: (0, 0)),
            pl.BlockSpec((1, NBT), lambda g: (0, 0)),
        ],
        out_specs=pl.BlockSpec((EBLK, NBT), lambda g: (g, 0)),
        out_shape=jax.ShapeDtypeStruct((E, NBT), _F32),
    )(iR, jR, dR, ef, er, t, Wc2, wd, bvec, W_b1, bb1)


def kernel(s, v, p, e, batch, edge_index_global,
           W_shared, b_shared, W_bond, b_bond,
           W_b0, b_b0, W_b1, b_b1,
           W_coords, W_atoms, b_atoms):
    try:
        info = plsc.get_sparse_core_info()
        nc, ns = info.num_cores, info.num_subcores
    except Exception:
        nc, ns = 2, 16

    j_ = edge_index_global[0].astype(jnp.int32)
    i_ = edge_index_global[1].astype(jnp.int32)
    batch2 = batch.astype(jnp.int32).reshape(1, N)
    vT = v.reshape(N, 3 * VDIM).T
    pT = p.T
    z = jnp.zeros((VDIM, 1), _F32)
    W3 = jnp.concatenate([
        jnp.concatenate([W_coords, z, z], axis=0),
        jnp.concatenate([z, W_coords, z], axis=0),
        jnp.concatenate([z, z, W_coords], axis=0),
    ], axis=1)
    W3T = W3.T
    W0 = W_b0[:SDIM]
    wd = W_b0[SDIM:SDIM + 1]
    bsh = b_shared.reshape(1, SDIM)
    bat = b_atoms.reshape(1, NAF)
    bb0 = b_b0.reshape(1, SDIM)
    bbond = b_bond.reshape(1, SDIM)
    bb1 = b_b1.reshape(1, NBT)
    epad = jnp.pad(e, ((0, PAD_ROWS), (0, 128 - EDIM)))

    t, atoms_pred, cpT, Wc2, bvec = _stage_a(
        s, vT, pT, batch2, W_shared, bsh, W3T, W_atoms, bat, W0, bb0,
        W_bond, bbond)

    T = _make_sc_table(nc, ns)(j_, i_)
    cpflat = cpT.reshape(3 * N)
    ef, er, dsq = _make_sc_gather(nc, ns)(T, j_, i_, epad, cpflat)

    iR = i_.reshape(NBLK, EBLK, 1)
    jR = j_.reshape(NBLK, EBLK, 1)
    dR = dsq.reshape(NBLK, EBLK, 1)
    ef2 = ef.reshape(E, EDIM)
    er2 = er.reshape(E, EDIM)
    bonds_pred = _stage_c(iR, jR, dR, ef2, er2, t, Wc2, wd, bvec, W_b1, bb1)

    coords_pred = cpT.T
    return coords_pred, atoms_pred, bonds_pred


# speedup vs baseline: 19.9738x; 1.2655x over previous
import functools

import jax
import jax.numpy as jnp
from jax import lax
from jax.experimental import pallas as pl
from jax.experimental.pallas import tpu as pltpu
from jax.experimental.pallas import tpu_sc as plsc

N = 1024
E = 65536
SDIM = 256
VDIM = 64
EDIM = 16
NAF = 16
NBT = 5
NG = 64

PAD_ROWS = 256
EBLK = 512
NBLK = E // EBLK
CH = 2048
SUB = 128

_F32 = jnp.float32
_HI = lax.Precision.HIGHEST


def _stage_a_body(s_ref, vT_ref, pT_ref, batch_ref, Wsh_ref, bsh_ref,
                  W3T_ref, Wat_ref, bat_ref, W0_ref, bb0_ref, Wbond_ref,
                  bbond_ref, t_ref, atoms_ref, cpT_ref, Wc2_ref, bvec_ref):
    s2 = jax.nn.silu(
        jnp.dot(s_ref[...], Wsh_ref[...], preferred_element_type=_F32,
                precision=_HI) + bsh_ref[...])
    t_ref[...] = jnp.dot(s2, W0_ref[...], preferred_element_type=_F32,
                         precision=_HI)
    atoms_ref[...] = jnp.dot(s2, Wat_ref[...], preferred_element_type=_F32,
                             precision=_HI) + bat_ref[...]
    cp0 = jnp.dot(W3T_ref[...], vT_ref[...], preferred_element_type=_F32,
                  precision=_HI) + pT_ref[...]
    bI = (lax.broadcasted_iota(jnp.int32, (NG, N), 0)
          == batch_ref[...]).astype(_F32)
    counts_row = lax.dot_general(jnp.ones((1, N), _F32), bI,
                                 (((1,), (1,)), ((), ())),
                                 preferred_element_type=_F32)
    segT = lax.dot_general(cp0, bI, (((1,), (1,)), ((), ())),
                           preferred_element_type=_F32, precision=_HI)
    meanT = segT / jnp.maximum(counts_row, 1.0)
    cpT_ref[...] = cp0 - jnp.dot(meanT, bI, preferred_element_type=_F32,
                                 precision=_HI)
    Wc2_ref[...] = 0.5 * jnp.dot(Wbond_ref[...], W0_ref[...],
                                 preferred_element_type=_F32, precision=_HI)
    bvec_ref[...] = jnp.dot(bbond_ref[...], W0_ref[...],
                            preferred_element_type=_F32,
                            precision=_HI) + bb0_ref[...]


def _stage_a(s, vT, pT, batch2, W_shared, bsh, W3T, W_atoms, bat, W0, bb0,
             W_bond, bbond):
    return pl.pallas_call(
        _stage_a_body,
        out_shape=[
            jax.ShapeDtypeStruct((N, SDIM), _F32),
            jax.ShapeDtypeStruct((N, NAF), _F32),
            jax.ShapeDtypeStruct((3, N), _F32),
            jax.ShapeDtypeStruct((EDIM, SDIM), _F32),
            jax.ShapeDtypeStruct((1, SDIM), _F32),
        ],
    )(s, vT, pT, batch2, W_shared, bsh, W3T, W_atoms, bat, W0, bb0, W_bond,
      bbond)


def _make_sc_table(nc, ns):
    nw = nc * ns
    spw = (N * N) // nw
    mesh = plsc.VectorSubcoreMesh(core_axis_name="c", subcore_axis_name="s",
                                  num_cores=nc, num_subcores=ns)

    @functools.partial(
        pl.kernel,
        out_type=jax.ShapeDtypeStruct((N * N,), jnp.int32),
        mesh=mesh,
        compiler_params=pltpu.CompilerParams(needs_layout_passes=False),
        scratch_types=[
            pltpu.VMEM((CH,), jnp.int32),
            pltpu.VMEM((CH,), jnp.int32),
            pltpu.VMEM((spw,), jnp.int32),
        ],
    )
    def sc_table(j_hbm, i_hbm, T_hbm, jv, iv, Lb):
        wid = lax.axis_index("s") * nc + lax.axis_index("c")
        lo = wid * spw

        def init(q, carry):
            Lb[pl.ds(q * 16, 16)] = jnp.full((16,), -1, jnp.int32)
            return carry
        lax.fori_loop(0, spw // 16, init, 0)

        def chunk(c, carry):
            pltpu.sync_copy(j_hbm.at[pl.ds(c * CH, CH)], jv)
            pltpu.sync_copy(i_hbm.at[pl.ds(c * CH, CH)], iv)

            def scan(q, carry2):
                j16 = jv[pl.ds(q * 16, 16)]
                i16 = iv[pl.ds(q * 16, 16)]
                key = j16 * N + i16
                loc = key - lo
                m = (loc >= 0) & (loc < spw)
                kvec = lax.iota(jnp.int32, 16) + (c * CH + q * 16)
                plsc.store_scatter(Lb, [loc], kvec, mask=m)
                return carry2
            lax.fori_loop(0, CH // 16, scan, 0)
            return carry
        lax.fori_loop(0, E // CH, chunk, 0)
        pltpu.sync_copy(Lb, T_hbm.at[pl.ds(lo, spw)])

    return sc_table


def _make_sc_gather(nc, ns):
    nw = nc * ns
    epw = E // nw
    nsub = epw // SUB
    mesh = plsc.VectorSubcoreMesh(core_axis_name="c", subcore_axis_name="s",
                                  num_cores=nc, num_subcores=ns)

    @functools.partial(
        pl.kernel,
        out_type=[
            jax.ShapeDtypeStruct((E * EDIM,), _F32),
            jax.ShapeDtypeStruct((E * EDIM,), _F32),
            jax.ShapeDtypeStruct((E,), _F32),
        ],
        mesh=mesh,
        compiler_params=pltpu.CompilerParams(needs_layout_passes=False),
        scratch_types=[
            pltpu.VMEM((epw,), jnp.int32),
            pltpu.VMEM((epw,), jnp.int32),
            pltpu.VMEM((epw,), jnp.int32),
            pltpu.VMEM((epw,), jnp.int32),
            pltpu.VMEM((epw,), jnp.int32),
            pltpu.VMEM((epw,), jnp.int32),
            pltpu.VMEM((epw * EDIM,), _F32),
            pltpu.VMEM((epw * EDIM,), _F32),
            pltpu.VMEM((SUB, 128), _F32),
            pltpu.VMEM((SUB, 128), _F32),
            pltpu.VMEM((epw,), _F32),
            pltpu.VMEM((N,), _F32),
            pltpu.VMEM((N,), _F32),
            pltpu.VMEM((N,), _F32),
            pltpu.SemaphoreType.DMA,
            pltpu.SemaphoreType.DMA,
        ],
    )
    def sc_gather(T_hbm, j_hbm, i_hbm, epad_hbm, cpT_hbm,
                  ef_hbm, er_hbm, d_hbm,
                  jv, iv, keyb, rkeyb, fidb, ridb, efb, erb,
                  slabA, slabB, db, cx, cy, cz, semA, semB):
        wid = lax.axis_index("s") * nc + lax.axis_index("c")
        base = wid * epw
        pltpu.sync_copy(j_hbm.at[pl.ds(base, epw)], jv)
        pltpu.sync_copy(i_hbm.at[pl.ds(base, epw)], iv)
        pltpu.sync_copy(cpT_hbm.at[pl.ds(0, N)], cx)
        pltpu.sync_copy(cpT_hbm.at[pl.ds(N, N)], cy)
        pltpu.sync_copy(cpT_hbm.at[pl.ds(2 * N, N)], cz)

        def phase1(q, carry):
            sl = pl.ds(q * 16, 16)
            j16 = jv[sl]
            i16 = iv[sl]
            keyb[sl] = j16 * N + i16
            rkeyb[sl] = i16 * N + j16
            xi = plsc.load_gather(cx, [i16])
            xj = plsc.load_gather(cx, [j16])
            yi = plsc.load_gather(cy, [i16])
            yj = plsc.load_gather(cy, [j16])
            zi = plsc.load_gather(cz, [i16])
            zj = plsc.load_gather(cz, [j16])
            dx = xi - xj
            dy = yi - yj
            dz = zi - zj
            db[sl] = dx * dx + dy * dy + dz * dz
            return carry
        lax.fori_loop(0, epw // 16, phase1, 0)

        cps = []
        for b in range(nsub):
            sl = pl.ds(b * SUB, SUB)
            cps.append(pltpu.async_copy(T_hbm.at[keyb.at[sl]], fidb.at[sl],
                                        semA))
        for cp in cps:
            cp.wait()
        cps = []
        for b in range(nsub):
            sl = pl.ds(b * SUB, SUB)
            cps.append(pltpu.async_copy(T_hbm.at[rkeyb.at[sl]], ridb.at[sl],
                                        semA))
        for cp in cps:
            cp.wait()

        def phase2(q, carry):
            sl = pl.ds(q * 16, 16)
            r16 = ridb[sl]
            pad = E + ((lax.iota(jnp.int32, 16) + (q * 16 + base))
                       & (PAD_ROWS - 1))
            ridb[sl] = jnp.where(r16 < 0, pad, r16)
            return carry
        lax.fori_loop(0, epw // 16, phase2, 0)

        def run_dir(idx_ref, out_ref):
            slabs = (slabA, slabB)
            sems = (semA, semB)
            cps = [None, None]
            cps[0] = pltpu.async_copy(epad_hbm.at[idx_ref.at[pl.ds(0, SUB)]],
                                      slabs[0], sems[0])
            for b in range(nsub):
                if b + 1 < nsub:
                    cps[(b + 1) % 2] = pltpu.async_copy(
                        epad_hbm.at[idx_ref.at[pl.ds((b + 1) * SUB, SUB)]],
                        slabs[(b + 1) % 2], sems[(b + 1) % 2])
                cps[b % 2].wait()
                slab = slabs[b % 2]

                def extract(r, carry):
                    out_ref[pl.ds((b * SUB + r) * 16, 16)] = slab[r, pl.ds(0, 16)]
                    return carry
                lax.fori_loop(0, SUB, extract, 0)

        run_dir(fidb, efb)
        run_dir(ridb, erb)

        pltpu.sync_copy(efb, ef_hbm.at[pl.ds(base * EDIM, epw * EDIM)])
        pltpu.sync_copy(erb, er_hbm.at[pl.ds(base * EDIM, epw * EDIM)])
        pltpu.sync_copy(db, d_hbm.at[pl.ds(base, epw)])

    return sc_gather


def _stage_c_body(iR, jR, dR, efR, erR, tR, Wc2R, wdR, bvecR, Wb1R, bb1R, oR):
    icol = iR[0]
    jcol = jR[0]
    ni = lax.broadcasted_iota(jnp.int32, (EBLK, N), 1)
    oh = (ni == icol).astype(_F32) + (ni == jcol).astype(_F32)
    h = jnp.dot(oh, tR[...], preferred_element_type=_F32)
    e2 = efR[...] + erR[...]
    h = h + jnp.dot(e2, Wc2R[...], preferred_element_type=_F32)
    h = h + dR[0] * wdR[...] + bvecR[...]
    sil = h * jax.nn.sigmoid(h)
    oR[...] = jnp.dot(sil, Wb1R[...], preferred_element_type=_F32) + bb1R[...]


def _stage_c(iR, jR, dR, ef, er, t, Wc2, wd, bvec, W_b1, bb1):
    return pl.pallas_call(
        _stage_c_body,
        grid=(NBLK,),
        in_specs=[
            pl.BlockSpec((1, EBLK, 1), lambda g: (g, 0, 0)),
            pl.BlockSpec((1, EBLK, 1), lambda g: (g, 0, 0)),
            pl.BlockSpec((1, EBLK, 1), lambda g: (g, 0, 0)),
            pl.BlockSpec((EBLK, EDIM), lambda g: (g, 0)),
            pl.BlockSpec((EBLK, EDIM), lambda g: (g, 0)),
            pl.BlockSpec((N, SDIM), lambda g: (0, 0)),
            pl.BlockSpec((EDIM, SDIM), lambda g: (0, 0)),
            pl.BlockSpec((1, SDIM), lambda g: (0, 0)),
            pl.BlockSpec((1, SDIM), lambda g: (0, 0)),
            pl.BlockSpec((SDIM, NBT), lambda g: (0, 0)),
            pl.BlockSpec((1, NBT), lambda g: (0, 0)),
        ],
        out_specs=pl.BlockSpec((EBLK, NBT), lambda g: (g, 0)),
        out_shape=jax.ShapeDtypeStruct((E, NBT), _F32),
    )(iR, jR, dR, ef, er, t, Wc2, wd, bvec, W_b1, bb1)


def kernel(s, v, p, e, batch, edge_index_global,
           W_shared, b_shared, W_bond, b_bond,
           W_b0, b_b0, W_b1, b_b1,
           W_coords, W_atoms, b_atoms):
    try:
        info = plsc.get_sparse_core_info()
        nc, ns = info.num_cores, info.num_subcores
    except Exception:
        nc, ns = 2, 16

    j_ = edge_index_global[0].astype(jnp.int32)
    i_ = edge_index_global[1].astype(jnp.int32)
    batch2 = batch.astype(jnp.int32).reshape(1, N)
    vT = v.reshape(N, 3 * VDIM).T
    pT = p.T
    z = jnp.zeros((VDIM, 1), _F32)
    W3 = jnp.concatenate([
        jnp.concatenate([W_coords, z, z], axis=0),
        jnp.concatenate([z, W_coords, z], axis=0),
        jnp.concatenate([z, z, W_coords], axis=0),
    ], axis=1)
    W3T = W3.T
    W0 = W_b0[:SDIM]
    wd = W_b0[SDIM:SDIM + 1]
    bsh = b_shared.reshape(1, SDIM)
    bat = b_atoms.reshape(1, NAF)
    bb0 = b_b0.reshape(1, SDIM)
    bbond = b_bond.reshape(1, SDIM)
    bb1 = b_b1.reshape(1, NBT)
    epad = jnp.pad(e, ((0, PAD_ROWS), (0, 128 - EDIM)))

    t, atoms_pred, cpT, Wc2, bvec = _stage_a(
        s, vT, pT, batch2, W_shared, bsh, W3T, W_atoms, bat, W0, bb0,
        W_bond, bbond)

    T = _make_sc_table(nc, ns)(j_, i_)
    cpflat = cpT.reshape(3 * N)
    ef, er, dsq = _make_sc_gather(nc, ns)(T, j_, i_, epad, cpflat)

    iR = i_.reshape(NBLK, EBLK, 1)
    jR = j_.reshape(NBLK, EBLK, 1)
    dR = dsq.reshape(NBLK, EBLK, 1)
    ef2 = ef.reshape(E, EDIM)
    er2 = er.reshape(E, EDIM)
    bonds_pred = _stage_c(iR, jR, dR, ef2, er2, t, Wc2, wd, bvec, W_b1, bb1)

    coords_pred = cpT.T
    return coords_pred, atoms_pred, bonds_pred


# speedup vs baseline: 21.2736x; 1.0651x over previous
import functools

import jax
import jax.numpy as jnp
from jax import lax
from jax.experimental import pallas as pl
from jax.experimental.pallas import tpu as pltpu
from jax.experimental.pallas import tpu_sc as plsc

N = 1024
E = 65536
SDIM = 256
VDIM = 64
EDIM = 16
NAF = 16
NBT = 5
NG = 64

PAD_ROWS = 256
EBLK = 512
NBLK = E // EBLK
CH = 2048
SUB = 128

_F32 = jnp.float32
_HI = lax.Precision.HIGHEST


def _stage_a_body(s_ref, vT_ref, pT_ref, batch_ref, Wsh_ref, bsh_ref,
                  W3T_ref, Wat_ref, bat_ref, W0_ref, bb0_ref, Wbond_ref,
                  bbond_ref, t_ref, atoms_ref, cpT_ref, Wc2_ref, bvec_ref):
    s2 = jax.nn.silu(
        jnp.dot(s_ref[...], Wsh_ref[...], preferred_element_type=_F32,
                precision=_HI) + bsh_ref[...])
    t_ref[...] = jnp.dot(s2, W0_ref[...], preferred_element_type=_F32,
                         precision=_HI)
    atoms_ref[...] = jnp.dot(s2, Wat_ref[...], preferred_element_type=_F32,
                             precision=_HI) + bat_ref[...]
    cp0 = jnp.dot(W3T_ref[...], vT_ref[...], preferred_element_type=_F32,
                  precision=_HI) + pT_ref[...]
    bI = (lax.broadcasted_iota(jnp.int32, (NG, N), 0)
          == batch_ref[...]).astype(_F32)
    counts_row = lax.dot_general(jnp.ones((1, N), _F32), bI,
                                 (((1,), (1,)), ((), ())),
                                 preferred_element_type=_F32)
    segT = lax.dot_general(cp0, bI, (((1,), (1,)), ((), ())),
                           preferred_element_type=_F32, precision=_HI)
    meanT = segT / jnp.maximum(counts_row, 1.0)
    cpT_ref[...] = cp0 - jnp.dot(meanT, bI, preferred_element_type=_F32,
                                 precision=_HI)
    Wc2_ref[...] = 0.5 * jnp.dot(Wbond_ref[...], W0_ref[...],
                                 preferred_element_type=_F32, precision=_HI)
    bvec_ref[...] = jnp.dot(bbond_ref[...], W0_ref[...],
                            preferred_element_type=_F32,
                            precision=_HI) + bb0_ref[...]


def _stage_a(s, vT, pT, batch2, W_shared, bsh, W3T, W_atoms, bat, W0, bb0,
             W_bond, bbond):
    return pl.pallas_call(
        _stage_a_body,
        out_shape=[
            jax.ShapeDtypeStruct((N, SDIM), _F32),
            jax.ShapeDtypeStruct((N, NAF), _F32),
            jax.ShapeDtypeStruct((3, N), _F32),
            jax.ShapeDtypeStruct((EDIM, SDIM), _F32),
            jax.ShapeDtypeStruct((1, SDIM), _F32),
        ],
    )(s, vT, pT, batch2, W_shared, bsh, W3T, W_atoms, bat, W0, bb0, W_bond,
      bbond)


def _make_sc_table(nc, ns):
    nw = nc * ns
    spw = (N * N) // nw
    mesh = plsc.VectorSubcoreMesh(core_axis_name="c", subcore_axis_name="s",
                                  num_cores=nc, num_subcores=ns)

    @functools.partial(
        pl.kernel,
        out_type=jax.ShapeDtypeStruct((N * N,), jnp.int32),
        mesh=mesh,
        compiler_params=pltpu.CompilerParams(needs_layout_passes=False),
        scratch_types=[
            pltpu.VMEM((CH,), jnp.int32),
            pltpu.VMEM((spw,), jnp.int32),
        ],
    )
    def sc_table(key_hbm, m1_hbm, T_hbm, kv, Lb):
        wid = lax.axis_index("s") * nc + lax.axis_index("c")
        lo = wid * spw
        pltpu.sync_copy(m1_hbm, Lb)

        lanes = lax.iota(jnp.int32, 16)

        def chunk(c, carry):
            pltpu.sync_copy(key_hbm.at[pl.ds(c * CH, CH)], kv)

            def scan(q, carry2):
                for u in range(4):
                    qq = q * 4 + u
                    k16 = kv[pl.ds(qq * 16, 16)]
                    loc = k16 - lo
                    m = (loc >= 0) & (loc < spw)
                    kvec = lanes + (c * CH + qq * 16)
                    plsc.store_scatter(Lb, [loc], kvec, mask=m)
                return carry2
            lax.fori_loop(0, CH // 64, scan, 0)
            return carry
        lax.fori_loop(0, E // CH, chunk, 0)
        pltpu.sync_copy(Lb, T_hbm.at[pl.ds(lo, spw)])

    return sc_table


def _make_sc_gather(nc, ns):
    nw = nc * ns
    epw = E // nw
    nsub = epw // SUB
    mesh = plsc.VectorSubcoreMesh(core_axis_name="c", subcore_axis_name="s",
                                  num_cores=nc, num_subcores=ns)

    @functools.partial(
        pl.kernel,
        out_type=[
            jax.ShapeDtypeStruct((E * EDIM,), _F32),
            jax.ShapeDtypeStruct((E * EDIM,), _F32),
            jax.ShapeDtypeStruct((E,), _F32),
        ],
        mesh=mesh,
        compiler_params=pltpu.CompilerParams(needs_layout_passes=False),
        scratch_types=[
            pltpu.VMEM((epw,), jnp.int32),
            pltpu.VMEM((epw,), jnp.int32),
            pltpu.VMEM((epw,), jnp.int32),
            pltpu.VMEM((epw,), jnp.int32),
            pltpu.VMEM((epw,), jnp.int32),
            pltpu.VMEM((epw,), jnp.int32),
            pltpu.VMEM((epw * EDIM,), _F32),
            pltpu.VMEM((epw * EDIM,), _F32),
            pltpu.VMEM((SUB, 128), _F32),
            pltpu.VMEM((SUB, 128), _F32),
            pltpu.VMEM((epw,), _F32),
            pltpu.VMEM((N,), _F32),
            pltpu.VMEM((N,), _F32),
            pltpu.VMEM((N,), _F32),
            pltpu.SemaphoreType.DMA,
            pltpu.SemaphoreType.DMA,
        ],
    )
    def sc_gather(T_hbm, j_hbm, i_hbm, keyf_hbm, keyr_hbm, epad_hbm, cpT_hbm,
                  ef_hbm, er_hbm, d_hbm,
                  jv, iv, keyb, rkeyb, fidb, ridb, efb, erb,
                  slabA, slabB, db, cx, cy, cz, semA, semB):
        wid = lax.axis_index("s") * nc + lax.axis_index("c")
        base = wid * epw
        pltpu.sync_copy(j_hbm.at[pl.ds(base, epw)], jv)
        pltpu.sync_copy(i_hbm.at[pl.ds(base, epw)], iv)
        pltpu.sync_copy(keyf_hbm.at[pl.ds(base, epw)], keyb)
        pltpu.sync_copy(keyr_hbm.at[pl.ds(base, epw)], rkeyb)
        pltpu.sync_copy(cpT_hbm.at[pl.ds(0, N)], cx)
        pltpu.sync_copy(cpT_hbm.at[pl.ds(N, N)], cy)
        pltpu.sync_copy(cpT_hbm.at[pl.ds(2 * N, N)], cz)

        cpsT = []
        for b in range(nsub):
            sl = pl.ds(b * SUB, SUB)
            cpsT.append(pltpu.async_copy(T_hbm.at[keyb.at[sl]], fidb.at[sl],
                                         semA))
        for b in range(nsub):
            sl = pl.ds(b * SUB, SUB)
            cpsT.append(pltpu.async_copy(T_hbm.at[rkeyb.at[sl]], ridb.at[sl],
                                         semA))

        def phase1(q, carry):
            sl = pl.ds(q * 16, 16)
            j16 = jv[sl]
            i16 = iv[sl]
            xi = plsc.load_gather(cx, [i16])
            xj = plsc.load_gather(cx, [j16])
            yi = plsc.load_gather(cy, [i16])
            yj = plsc.load_gather(cy, [j16])
            zi = plsc.load_gather(cz, [i16])
            zj = plsc.load_gather(cz, [j16])
            dx = xi - xj
            dy = yi - yj
            dz = zi - zj
            db[sl] = dx * dx + dy * dy + dz * dz
            return carry
        lax.fori_loop(0, epw // 16, phase1, 0)
        pltpu.sync_copy(db, d_hbm.at[pl.ds(base, epw)])

        for cp in cpsT:
            cp.wait()

        def phase2(q, carry):
            sl = pl.ds(q * 16, 16)
            r16 = ridb[sl]
            pad = E + ((lax.iota(jnp.int32, 16) + (q * 16 + base))
                       & (PAD_ROWS - 1))
            ridb[sl] = jnp.where(r16 < 0, pad, r16)
            return carry
        lax.fori_loop(0, epw // 16, phase2, 0)

        def run_dir(idx_ref, out_ref):
            slabs = (slabA, slabB)
            sems = (semA, semB)
            cps = [None, None]
            cps[0] = pltpu.async_copy(epad_hbm.at[idx_ref.at[pl.ds(0, SUB)]],
                                      slabs[0], sems[0])
            for b in range(nsub):
                if b + 1 < nsub:
                    cps[(b + 1) % 2] = pltpu.async_copy(
                        epad_hbm.at[idx_ref.at[pl.ds((b + 1) * SUB, SUB)]],
                        slabs[(b + 1) % 2], sems[(b + 1) % 2])
                cps[b % 2].wait()
                slab = slabs[b % 2]

                def extract(r, carry):
                    out_ref[pl.ds((b * SUB + r) * 16, 16)] = \
                        slab[r, pl.ds(0, 16)]
                    return carry
                lax.fori_loop(0, SUB, extract, 0)

        run_dir(fidb, efb)
        run_dir(ridb, erb)

        pltpu.sync_copy(efb, ef_hbm.at[pl.ds(base * EDIM, epw * EDIM)])
        pltpu.sync_copy(erb, er_hbm.at[pl.ds(base * EDIM, epw * EDIM)])

    return sc_gather


def _stage_c_body(iR, jR, dR, efR, erR, tR, Wc2R, wdR, bvecR, Wb1R, bb1R, oR):
    icol = iR[0]
    jcol = jR[0]
    ni = lax.broadcasted_iota(jnp.int32, (EBLK, N), 1)
    oh = (ni == icol).astype(_F32) + (ni == jcol).astype(_F32)
    h = jnp.dot(oh, tR[...], preferred_element_type=_F32)
    e2 = efR[...] + erR[...]
    h = h + jnp.dot(e2, Wc2R[...], preferred_element_type=_F32)
    h = h + dR[0] * wdR[...] + bvecR[...]
    sil = h * jax.nn.sigmoid(h)
    oR[...] = jnp.dot(sil, Wb1R[...], preferred_element_type=_F32) + bb1R[...]


def _stage_c(iR, jR, dR, ef, er, t, Wc2, wd, bvec, W_b1, bb1):
    return pl.pallas_call(
        _stage_c_body,
        grid=(NBLK,),
        in_specs=[
            pl.BlockSpec((1, EBLK, 1), lambda g: (g, 0, 0)),
            pl.BlockSpec((1, EBLK, 1), lambda g: (g, 0, 0)),
            pl.BlockSpec((1, EBLK, 1), lambda g: (g, 0, 0)),
            pl.BlockSpec((EBLK, EDIM), lambda g: (g, 0)),
            pl.BlockSpec((EBLK, EDIM), lambda g: (g, 0)),
            pl.BlockSpec((N, SDIM), lambda g: (0, 0)),
            pl.BlockSpec((EDIM, SDIM), lambda g: (0, 0)),
            pl.BlockSpec((1, SDIM), lambda g: (0, 0)),
            pl.BlockSpec((1, SDIM), lambda g: (0, 0)),
            pl.BlockSpec((SDIM, NBT), lambda g: (0, 0)),
            pl.BlockSpec((1, NBT), lambda g: (0, 0)),
        ],
        out_specs=pl.BlockSpec((EBLK, NBT), lambda g: (g, 0)),
        out_shape=jax.ShapeDtypeStruct((E, NBT), _F32),
    )(iR, jR, dR, ef, er, t, Wc2, wd, bvec, W_b1, bb1)


def kernel(s, v, p, e, batch, edge_index_global,
           W_shared, b_shared, W_bond, b_bond,
           W_b0, b_b0, W_b1, b_b1,
           W_coords, W_atoms, b_atoms):
    try:
        info = plsc.get_sparse_core_info()
        nc, ns = info.num_cores, info.num_subcores
    except Exception:
        nc, ns = 2, 16

    j_ = edge_index_global[0].astype(jnp.int32)
    i_ = edge_index_global[1].astype(jnp.int32)
    batch2 = batch.astype(jnp.int32).reshape(1, N)
    vT = v.reshape(N, 3 * VDIM).T
    pT = p.T
    z = jnp.zeros((VDIM, 1), _F32)
    W3 = jnp.concatenate([
        jnp.concatenate([W_coords, z, z], axis=0),
        jnp.concatenate([z, W_coords, z], axis=0),
        jnp.concatenate([z, z, W_coords], axis=0),
    ], axis=1)
    W3T = W3.T
    W0 = W_b0[:SDIM]
    wd = W_b0[SDIM:SDIM + 1]
    bsh = b_shared.reshape(1, SDIM)
    bat = b_atoms.reshape(1, NAF)
    bb0 = b_b0.reshape(1, SDIM)
    bbond = b_bond.reshape(1, SDIM)
    bb1 = b_b1.reshape(1, NBT)
    epad = jnp.pad(e, ((0, PAD_ROWS), (0, 128 - EDIM)))

    t, atoms_pred, cpT, Wc2, bvec = _stage_a(
        s, vT, pT, batch2, W_shared, bsh, W3T, W_atoms, bat, W0, bb0,
        W_bond, bbond)

    keyf = j_ * N + i_
    keyr = i_ * N + j_
    m1 = jnp.full(((N * N) // (nc * ns),), -1, jnp.int32)
    T = _make_sc_table(nc, ns)(keyf, m1)
    cpflat = cpT.reshape(3 * N)
    ef, er, dsq = _make_sc_gather(nc, ns)(T, j_, i_, keyf, keyr, epad,
                                          cpflat)

    iR = i_.reshape(NBLK, EBLK, 1)
    jR = j_.reshape(NBLK, EBLK, 1)
    dR = dsq.reshape(NBLK, EBLK, 1)
    ef2 = ef.reshape(E, EDIM)
    er2 = er.reshape(E, EDIM)
    bonds_pred = _stage_c(iR, jR, dR, ef2, er2, t, Wc2, wd, bvec, W_b1, bb1)

    coords_pred = cpT.T
    return coords_pred, atoms_pred, bonds_pred


# speedup vs baseline: 27.4757x; 1.2915x over previous
import functools

import jax
import jax.numpy as jnp
from jax import lax
from jax.experimental import pallas as pl
from jax.experimental.pallas import tpu as pltpu
from jax.experimental.pallas import tpu_sc as plsc

N = 1024
E = 65536
SDIM = 256
VDIM = 64
EDIM = 16
NAF = 16
NBT = 5
NG = 64

PAD_ROWS = 256
EBLK = 512
NBLK = E // EBLK
CH = 2048
SUB = 128

_F32 = jnp.float32
_HI = lax.Precision.HIGHEST


def _stage_a_body(s_ref, vT_ref, pT_ref, batch_ref, Wsh_ref, bsh_ref,
                  W3T_ref, Wat_ref, bat_ref, W0_ref, bb0_ref, Wbond_ref,
                  bbond_ref, t_ref, atoms_ref, cpT_ref, Wc2_ref, bvec_ref):
    s2 = jax.nn.silu(
        jnp.dot(s_ref[...], Wsh_ref[...], preferred_element_type=_F32,
                precision=_HI) + bsh_ref[...])
    t_ref[...] = jnp.dot(s2, W0_ref[...], preferred_element_type=_F32,
                         precision=_HI)
    atoms_ref[...] = jnp.dot(s2, Wat_ref[...], preferred_element_type=_F32,
                             precision=_HI) + bat_ref[...]
    cp0 = jnp.dot(W3T_ref[...], vT_ref[...], preferred_element_type=_F32,
                  precision=_HI) + pT_ref[...]
    bI = (lax.broadcasted_iota(jnp.int32, (NG, N), 0)
          == batch_ref[...]).astype(_F32)
    counts_row = lax.dot_general(jnp.ones((1, N), _F32), bI,
                                 (((1,), (1,)), ((), ())),
                                 preferred_element_type=_F32)
    segT = lax.dot_general(cp0, bI, (((1,), (1,)), ((), ())),
                           preferred_element_type=_F32, precision=_HI)
    meanT = segT / jnp.maximum(counts_row, 1.0)
    cpT_ref[...] = cp0 - jnp.dot(meanT, bI, preferred_element_type=_F32,
                                 precision=_HI)
    Wc2_ref[...] = 0.5 * jnp.dot(Wbond_ref[...], W0_ref[...],
                                 preferred_element_type=_F32, precision=_HI)
    bvec_ref[...] = jnp.dot(bbond_ref[...], W0_ref[...],
                            preferred_element_type=_F32,
                            precision=_HI) + bb0_ref[...]


def _stage_a(s, vT, pT, batch2, W_shared, bsh, W3T, W_atoms, bat, W0, bb0,
             W_bond, bbond):
    return pl.pallas_call(
        _stage_a_body,
        out_shape=[
            jax.ShapeDtypeStruct((N, SDIM), _F32),
            jax.ShapeDtypeStruct((N, NAF), _F32),
            jax.ShapeDtypeStruct((3, N), _F32),
            jax.ShapeDtypeStruct((EDIM, SDIM), _F32),
            jax.ShapeDtypeStruct((1, SDIM), _F32),
        ],
    )(s, vT, pT, batch2, W_shared, bsh, W3T, W_atoms, bat, W0, bb0, W_bond,
      bbond)


def _make_sc_table(nc, ns):
    nw = nc * ns
    spw = (N * N) // nw
    mesh = plsc.VectorSubcoreMesh(core_axis_name="c", subcore_axis_name="s",
                                  num_cores=nc, num_subcores=ns)

    @functools.partial(
        pl.kernel,
        out_type=jax.ShapeDtypeStruct((N * N,), jnp.int32),
        mesh=mesh,
        compiler_params=pltpu.CompilerParams(needs_layout_passes=False),
        scratch_types=[
            pltpu.VMEM((CH,), jnp.int32),
            pltpu.VMEM((spw,), jnp.int32),
        ],
    )
    def sc_table(key_hbm, m1_hbm, T_hbm, kv, Lb):
        wid = lax.axis_index("s") * nc + lax.axis_index("c")
        lo = wid * spw
        pltpu.sync_copy(m1_hbm, Lb)

        lanes = lax.iota(jnp.int32, 16)

        def chunk(c, carry):
            pltpu.sync_copy(key_hbm.at[pl.ds(c * CH, CH)], kv)

            def scan(q, carry2):
                for u in range(4):
                    qq = q * 4 + u
                    k16 = kv[pl.ds(qq * 16, 16)]
                    loc = k16 - lo
                    m = (loc >= 0) & (loc < spw)
                    kvec = lanes + (c * CH + qq * 16)
                    plsc.store_scatter(Lb, [loc], kvec, mask=m)
                return carry2
            lax.fori_loop(0, CH // 64, scan, 0)
            return carry
        lax.fori_loop(0, E // CH, chunk, 0)
        pltpu.sync_copy(Lb, T_hbm.at[pl.ds(lo, spw)])

    return sc_table


def _make_sc_gather(nc, ns):
    nw = nc * ns
    epw = E // nw
    nsub = epw // SUB
    mesh = plsc.VectorSubcoreMesh(core_axis_name="c", subcore_axis_name="s",
                                  num_cores=nc, num_subcores=ns)

    @functools.partial(
        pl.kernel,
        out_type=[
            jax.ShapeDtypeStruct((E * EDIM,), _F32),
            jax.ShapeDtypeStruct((E,), _F32),
        ],
        mesh=mesh,
        compiler_params=pltpu.CompilerParams(needs_layout_passes=False),
        scratch_types=[
            pltpu.VMEM((epw,), jnp.int32),
            pltpu.VMEM((epw,), jnp.int32),
            pltpu.VMEM((epw,), jnp.int32),
            pltpu.VMEM((epw,), jnp.int32),
            pltpu.VMEM((epw,), jnp.int32),
            pltpu.VMEM((epw,), jnp.int32),
            pltpu.VMEM((epw * EDIM,), _F32),
            pltpu.VMEM((epw * EDIM,), _F32),
            pltpu.VMEM((SUB, 128), _F32),
            pltpu.VMEM((SUB, 128), _F32),
            pltpu.VMEM((epw,), _F32),
            pltpu.VMEM((N,), _F32),
            pltpu.VMEM((N,), _F32),
            pltpu.VMEM((N,), _F32),
            pltpu.SemaphoreType.DMA,
            pltpu.SemaphoreType.DMA,
        ],
    )
    def sc_gather(T_hbm, j_hbm, i_hbm, keyf_hbm, keyr_hbm, epad_hbm, cpT_hbm,
                  es_hbm, d_hbm,
                  jv, iv, keyb, rkeyb, fidb, ridb, efb, erb,
                  slabA, slabB, db, cx, cy, cz, semA, semB):
        wid = lax.axis_index("s") * nc + lax.axis_index("c")
        base = wid * epw
        pltpu.sync_copy(j_hbm.at[pl.ds(base, epw)], jv)
        pltpu.sync_copy(i_hbm.at[pl.ds(base, epw)], iv)
        pltpu.sync_copy(keyf_hbm.at[pl.ds(base, epw)], keyb)
        pltpu.sync_copy(keyr_hbm.at[pl.ds(base, epw)], rkeyb)
        pltpu.sync_copy(cpT_hbm.at[pl.ds(0, N)], cx)
        pltpu.sync_copy(cpT_hbm.at[pl.ds(N, N)], cy)
        pltpu.sync_copy(cpT_hbm.at[pl.ds(2 * N, N)], cz)

        cpsT = []
        for b in range(nsub):
            sl = pl.ds(b * SUB, SUB)
            cpsT.append(pltpu.async_copy(T_hbm.at[keyb.at[sl]], fidb.at[sl],
                                         semA))
        for b in range(nsub):
            sl = pl.ds(b * SUB, SUB)
            cpsT.append(pltpu.async_copy(T_hbm.at[rkeyb.at[sl]], ridb.at[sl],
                                         semA))

        def phase1(q, carry):
            sl = pl.ds(q * 16, 16)
            j16 = jv[sl]
            i16 = iv[sl]
            xi = plsc.load_gather(cx, [i16])
            xj = plsc.load_gather(cx, [j16])
            yi = plsc.load_gather(cy, [i16])
            yj = plsc.load_gather(cy, [j16])
            zi = plsc.load_gather(cz, [i16])
            zj = plsc.load_gather(cz, [j16])
            dx = xi - xj
            dy = yi - yj
            dz = zi - zj
            db[sl] = dx * dx + dy * dy + dz * dz
            return carry
        lax.fori_loop(0, epw // 16, phase1, 0)
        pltpu.sync_copy(db, d_hbm.at[pl.ds(base, epw)])

        for cp in cpsT:
            cp.wait()

        def phase2(q, carry):
            sl = pl.ds(q * 16, 16)
            r16 = ridb[sl]
            pad = E + ((lax.iota(jnp.int32, 16) + (q * 16 + base))
                       & (PAD_ROWS - 1))
            ridb[sl] = jnp.where(r16 < 0, pad, r16)
            return carry
        lax.fori_loop(0, epw // 16, phase2, 0)

        def run_dir(idx_ref, out_ref, accumulate):
            slabs = (slabA, slabB)
            sems = (semA, semB)
            cps = [None, None]
            cps[0] = pltpu.async_copy(epad_hbm.at[idx_ref.at[pl.ds(0, SUB)]],
                                      slabs[0], sems[0])
            for b in range(nsub):
                if b + 1 < nsub:
                    cps[(b + 1) % 2] = pltpu.async_copy(
                        epad_hbm.at[idx_ref.at[pl.ds((b + 1) * SUB, SUB)]],
                        slabs[(b + 1) % 2], sems[(b + 1) % 2])
                cps[b % 2].wait()
                slab = slabs[b % 2]

                def extract(r, carry):
                    sl16 = pl.ds((b * SUB + r) * 16, 16)
                    row = slab[r, pl.ds(0, 16)]
                    if accumulate:
                        row = row + out_ref[sl16]
                    out_ref[sl16] = row
                    return carry
                lax.fori_loop(0, SUB, extract, 0)

        run_dir(fidb, efb, False)
        run_dir(ridb, efb, True)

        pltpu.sync_copy(efb, es_hbm.at[pl.ds(base * EDIM, epw * EDIM)])

    return sc_gather


def _stage_c_body(iR, jR, dR, esR, tR, Wc2R, wdR, bvecR, Wb1R, bb1R, oR):
    irow = iR[0]
    jrow = jR[0]
    nn = lax.broadcasted_iota(jnp.int32, (N, EBLK), 0)
    ohT = (nn == irow).astype(_F32) + (nn == jrow).astype(_F32)
    tn = (((0,), (0,)), ((), ()))
    h = lax.dot_general(ohT, tR[...], tn, preferred_element_type=_F32)
    h = h + jnp.dot(esR[...], Wc2R[...], preferred_element_type=_F32)
    h = h + lax.dot_general(dR[0], wdR[...], tn,
                            preferred_element_type=_F32)
    h = h + bvecR[...]
    sil = h * jax.nn.sigmoid(h)
    oR[...] = jnp.dot(sil, Wb1R[...], preferred_element_type=_F32) + bb1R[...]


def _stage_c(iR, jR, dR, es, t, Wc2, wd, bvec, W_b1, bb1):
    return pl.pallas_call(
        _stage_c_body,
        grid=(NBLK,),
        in_specs=[
            pl.BlockSpec((1, 1, EBLK), lambda g: (g, 0, 0)),
            pl.BlockSpec((1, 1, EBLK), lambda g: (g, 0, 0)),
            pl.BlockSpec((1, 1, EBLK), lambda g: (g, 0, 0)),
            pl.BlockSpec((EBLK, EDIM), lambda g: (g, 0)),
            pl.BlockSpec((N, SDIM), lambda g: (0, 0)),
            pl.BlockSpec((EDIM, SDIM), lambda g: (0, 0)),
            pl.BlockSpec((1, SDIM), lambda g: (0, 0)),
            pl.BlockSpec((1, SDIM), lambda g: (0, 0)),
            pl.BlockSpec((SDIM, NBT), lambda g: (0, 0)),
            pl.BlockSpec((1, NBT), lambda g: (0, 0)),
        ],
        out_specs=pl.BlockSpec((EBLK, NBT), lambda g: (g, 0)),
        out_shape=jax.ShapeDtypeStruct((E, NBT), _F32),
    )(iR, jR, dR, es, t, Wc2, wd, bvec, W_b1, bb1)


def kernel(s, v, p, e, batch, edge_index_global,
           W_shared, b_shared, W_bond, b_bond,
           W_b0, b_b0, W_b1, b_b1,
           W_coords, W_atoms, b_atoms):
    try:
        info = plsc.get_sparse_core_info()
        nc, ns = info.num_cores, info.num_subcores
    except Exception:
        nc, ns = 2, 16

    j_ = edge_index_global[0].astype(jnp.int32)
    i_ = edge_index_global[1].astype(jnp.int32)
    batch2 = batch.astype(jnp.int32).reshape(1, N)
    vT = v.reshape(N, 3 * VDIM).T
    pT = p.T
    z = jnp.zeros((VDIM, 1), _F32)
    W3 = jnp.concatenate([
        jnp.concatenate([W_coords, z, z], axis=0),
        jnp.concatenate([z, W_coords, z], axis=0),
        jnp.concatenate([z, z, W_coords], axis=0),
    ], axis=1)
    W3T = W3.T
    W0 = W_b0[:SDIM]
    wd = W_b0[SDIM:SDIM + 1]
    bsh = b_shared.reshape(1, SDIM)
    bat = b_atoms.reshape(1, NAF)
    bb0 = b_b0.reshape(1, SDIM)
    bbond = b_bond.reshape(1, SDIM)
    bb1 = b_b1.reshape(1, NBT)
    epad = jnp.pad(e, ((0, PAD_ROWS), (0, 128 - EDIM)))

    t, atoms_pred, cpT, Wc2, bvec = _stage_a(
        s, vT, pT, batch2, W_shared, bsh, W3T, W_atoms, bat, W0, bb0,
        W_bond, bbond)

    keyf = j_ * N + i_
    keyr = i_ * N + j_
    m1 = jnp.full(((N * N) // (nc * ns),), -1, jnp.int32)
    T = _make_sc_table(nc, ns)(keyf, m1)
    cpflat = cpT.reshape(3 * N)
    es, dsq = _make_sc_gather(nc, ns)(T, j_, i_, keyf, keyr, epad, cpflat)

    iR = i_.reshape(NBLK, 1, EBLK)
    jR = j_.reshape(NBLK, 1, EBLK)
    dR = dsq.reshape(NBLK, 1, EBLK)
    es2 = es.reshape(E, EDIM)
    bonds_pred = _stage_c(iR, jR, dR, es2, t, Wc2, wd, bvec, W_b1, bb1)

    coords_pred = cpT.T
    return coords_pred, atoms_pred, bonds_pred


# speedup vs baseline: 29.1742x; 1.0618x over previous
import functools

import jax
import jax.numpy as jnp
from jax import lax
from jax.experimental import pallas as pl
from jax.experimental.pallas import tpu as pltpu
from jax.experimental.pallas import tpu_sc as plsc

N = 1024
E = 65536
SDIM = 256
VDIM = 64
EDIM = 16
NAF = 16
NBT = 5
NG = 64

PAD_ROWS = 256
EBLK = 512
NBLK = E // EBLK
CH = 2048
SUB = 128

_F32 = jnp.float32
_HI = lax.Precision.HIGHEST


def _stage_a_body(s_ref, vT_ref, pT_ref, batch_ref, Wsh_ref, bsh_ref,
                  W3T_ref, Wat_ref, bat_ref, W0_ref, bb0_ref, Wbond_ref,
                  bbond_ref, t_ref, atoms_ref, cpT_ref, Wc2_ref, bvec_ref):
    s2 = jax.nn.silu(
        jnp.dot(s_ref[...], Wsh_ref[...], preferred_element_type=_F32,
                precision=_HI) + bsh_ref[...])
    t_ref[...] = jnp.dot(s2, W0_ref[...], preferred_element_type=_F32,
                         precision=_HI)
    atoms_ref[...] = jnp.dot(s2, Wat_ref[...], preferred_element_type=_F32,
                             precision=_HI) + bat_ref[...]
    cp0 = jnp.dot(W3T_ref[...], vT_ref[...], preferred_element_type=_F32,
                  precision=_HI) + pT_ref[...]
    bI = (lax.broadcasted_iota(jnp.int32, (NG, N), 0)
          == batch_ref[...]).astype(_F32)
    counts_row = lax.dot_general(jnp.ones((1, N), _F32), bI,
                                 (((1,), (1,)), ((), ())),
                                 preferred_element_type=_F32)
    segT = lax.dot_general(cp0, bI, (((1,), (1,)), ((), ())),
                           preferred_element_type=_F32, precision=_HI)
    meanT = segT / jnp.maximum(counts_row, 1.0)
    cpT_ref[...] = cp0 - jnp.dot(meanT, bI, preferred_element_type=_F32,
                                 precision=_HI)
    Wc2_ref[...] = 0.5 * jnp.dot(Wbond_ref[...], W0_ref[...],
                                 preferred_element_type=_F32, precision=_HI)
    bvec_ref[...] = jnp.dot(bbond_ref[...], W0_ref[...],
                            preferred_element_type=_F32,
                            precision=_HI) + bb0_ref[...]


def _stage_a(s, vT, pT, batch2, W_shared, bsh, W3T, W_atoms, bat, W0, bb0,
             W_bond, bbond):
    return pl.pallas_call(
        _stage_a_body,
        out_shape=[
            jax.ShapeDtypeStruct((N, SDIM), _F32),
            jax.ShapeDtypeStruct((N, NAF), _F32),
            jax.ShapeDtypeStruct((3, N), _F32),
            jax.ShapeDtypeStruct((EDIM, SDIM), _F32),
            jax.ShapeDtypeStruct((1, SDIM), _F32),
        ],
    )(s, vT, pT, batch2, W_shared, bsh, W3T, W_atoms, bat, W0, bb0, W_bond,
      bbond)


def _make_sc_table(nc, ns):
    nw = nc * ns
    spw = (N * N) // nw
    mesh = plsc.VectorSubcoreMesh(core_axis_name="c", subcore_axis_name="s",
                                  num_cores=nc, num_subcores=ns)

    @functools.partial(
        pl.kernel,
        out_type=jax.ShapeDtypeStruct((N * N,), jnp.int32),
        mesh=mesh,
        compiler_params=pltpu.CompilerParams(needs_layout_passes=False),
        scratch_types=[
            pltpu.VMEM((CH,), jnp.int32),
            pltpu.VMEM((CH,), jnp.int32),
            pltpu.VMEM((spw,), jnp.int32),
            pltpu.SemaphoreType.DMA,
            pltpu.SemaphoreType.DMA,
        ],
    )
    def sc_table(key_hbm, m1_hbm, T_hbm, kvA, kvB, Lb, semA, semB):
        wid = lax.axis_index("s") * nc + lax.axis_index("c")
        lo = wid * spw
        pltpu.sync_copy(m1_hbm, Lb)

        lanes = lax.iota(jnp.int32, 16)
        kvs = (kvA, kvB)
        sems = (semA, semB)
        nch = E // CH
        cps = [None, None]
        cps[0] = pltpu.async_copy(key_hbm.at[pl.ds(0, CH)], kvA, semA)
        for c in range(nch):
            if c + 1 < nch:
                cps[(c + 1) % 2] = pltpu.async_copy(
                    key_hbm.at[pl.ds((c + 1) * CH, CH)],
                    kvs[(c + 1) % 2], sems[(c + 1) % 2])
            cps[c % 2].wait()
            kv = kvs[c % 2]

            def scan(q, carry2):
                for u in range(4):
                    qq = q * 4 + u
                    k16 = kv[pl.ds(qq * 16, 16)]
                    loc = k16 - lo
                    m = (loc >= 0) & (loc < spw)
                    kvec = lanes + (c * CH + qq * 16)
                    plsc.store_scatter(Lb, [loc], kvec, mask=m)
                return carry2
            lax.fori_loop(0, CH // 64, scan, 0)
        pltpu.sync_copy(Lb, T_hbm.at[pl.ds(lo, spw)])

    return sc_table


def _make_sc_gather(nc, ns):
    nw = nc * ns
    epw = E // nw
    nsub = epw // SUB
    mesh = plsc.VectorSubcoreMesh(core_axis_name="c", subcore_axis_name="s",
                                  num_cores=nc, num_subcores=ns)

    @functools.partial(
        pl.kernel,
        out_type=[
            jax.ShapeDtypeStruct((E * EDIM,), _F32),
            jax.ShapeDtypeStruct((E,), _F32),
        ],
        mesh=mesh,
        compiler_params=pltpu.CompilerParams(needs_layout_passes=False),
        scratch_types=[
            pltpu.VMEM((epw,), jnp.int32),
            pltpu.VMEM((epw,), jnp.int32),
            pltpu.VMEM((epw,), jnp.int32),
            pltpu.VMEM((epw,), jnp.int32),
            pltpu.VMEM((epw,), jnp.int32),
            pltpu.VMEM((epw,), jnp.int32),
            pltpu.VMEM((epw * EDIM,), _F32),
            pltpu.VMEM((epw * EDIM,), _F32),
            pltpu.VMEM((SUB, 128), _F32),
            pltpu.VMEM((SUB, 128), _F32),
            pltpu.VMEM((epw,), _F32),
            pltpu.VMEM((N,), _F32),
            pltpu.VMEM((N,), _F32),
            pltpu.VMEM((N,), _F32),
            pltpu.SemaphoreType.DMA,
            pltpu.SemaphoreType.DMA,
        ],
    )
    def sc_gather(T_hbm, j_hbm, i_hbm, keyf_hbm, keyr_hbm, epad_hbm, cpT_hbm,
                  es_hbm, d_hbm,
                  jv, iv, keyb, rkeyb, fidb, ridb, efb, erb,
                  slabA, slabB, db, cx, cy, cz, semA, semB):
        wid = lax.axis_index("s") * nc + lax.axis_index("c")
        base = wid * epw
        pltpu.sync_copy(j_hbm.at[pl.ds(base, epw)], jv)
        pltpu.sync_copy(i_hbm.at[pl.ds(base, epw)], iv)
        pltpu.sync_copy(keyf_hbm.at[pl.ds(base, epw)], keyb)
        pltpu.sync_copy(keyr_hbm.at[pl.ds(base, epw)], rkeyb)
        pltpu.sync_copy(cpT_hbm.at[pl.ds(0, N)], cx)
        pltpu.sync_copy(cpT_hbm.at[pl.ds(N, N)], cy)
        pltpu.sync_copy(cpT_hbm.at[pl.ds(2 * N, N)], cz)

        cpsT = []
        for b in range(nsub):
            sl = pl.ds(b * SUB, SUB)
            cpsT.append(pltpu.async_copy(T_hbm.at[keyb.at[sl]], fidb.at[sl],
                                         semA))
        for b in range(nsub):
            sl = pl.ds(b * SUB, SUB)
            cpsT.append(pltpu.async_copy(T_hbm.at[rkeyb.at[sl]], ridb.at[sl],
                                         semA))

        def phase1(q, carry):
            sl = pl.ds(q * 16, 16)
            j16 = jv[sl]
            i16 = iv[sl]
            xi = plsc.load_gather(cx, [i16])
            xj = plsc.load_gather(cx, [j16])
            yi = plsc.load_gather(cy, [i16])
            yj = plsc.load_gather(cy, [j16])
            zi = plsc.load_gather(cz, [i16])
            zj = plsc.load_gather(cz, [j16])
            dx = xi - xj
            dy = yi - yj
            dz = zi - zj
            db[sl] = dx * dx + dy * dy + dz * dz
            return carry
        lax.fori_loop(0, epw // 16, phase1, 0)
        pltpu.sync_copy(db, d_hbm.at[pl.ds(base, epw)])

        for cp in cpsT:
            cp.wait()

        def phase2(q, carry):
            sl = pl.ds(q * 16, 16)
            r16 = ridb[sl]
            pad = E + ((lax.iota(jnp.int32, 16) + (q * 16 + base))
                       & (PAD_ROWS - 1))
            ridb[sl] = jnp.where(r16 < 0, pad, r16)
            return carry
        lax.fori_loop(0, epw // 16, phase2, 0)

        def run_dir(idx_ref, out_ref, accumulate):
            slabs = (slabA, slabB)
            sems = (semA, semB)
            cps = [None, None]
            cps[0] = pltpu.async_copy(epad_hbm.at[idx_ref.at[pl.ds(0, SUB)]],
                                      slabs[0], sems[0])
            for b in range(nsub):
                if b + 1 < nsub:
                    cps[(b + 1) % 2] = pltpu.async_copy(
                        epad_hbm.at[idx_ref.at[pl.ds((b + 1) * SUB, SUB)]],
                        slabs[(b + 1) % 2], sems[(b + 1) % 2])
                cps[b % 2].wait()
                slab = slabs[b % 2]

                def extract(r, carry):
                    sl16 = pl.ds((b * SUB + r) * 16, 16)
                    row = slab[r, pl.ds(0, 16)]
                    if accumulate:
                        row = row + out_ref[sl16]
                    out_ref[sl16] = row
                    return carry
                lax.fori_loop(0, SUB, extract, 0)

        run_dir(fidb, efb, False)
        run_dir(ridb, efb, True)

        pltpu.sync_copy(efb, es_hbm.at[pl.ds(base * EDIM, epw * EDIM)])

    return sc_gather


def _stage_c_body(iR, jR, dR, esR, tR, Wc2R, wdR, bvecR, Wb1R, bb1R, oR):
    irow = iR[0]
    jrow = jR[0]
    nn = lax.broadcasted_iota(jnp.int32, (N, EBLK), 0)
    ohT = (nn == irow).astype(_F32) + (nn == jrow).astype(_F32)
    tn = (((0,), (0,)), ((), ()))
    h = lax.dot_general(ohT, tR[...], tn, preferred_element_type=_F32)
    h = h + jnp.dot(esR[...], Wc2R[...], preferred_element_type=_F32)
    h = h + lax.dot_general(dR[0], wdR[...], tn,
                            preferred_element_type=_F32)
    h = h + bvecR[...]
    sil = h * jax.nn.sigmoid(h)
    oR[...] = jnp.dot(sil, Wb1R[...], preferred_element_type=_F32) + bb1R[...]


def _stage_c(iR, jR, dR, es, t, Wc2, wd, bvec, W_b1, bb1):
    return pl.pallas_call(
        _stage_c_body,
        grid=(NBLK,),
        in_specs=[
            pl.BlockSpec((1, 1, EBLK), lambda g: (g, 0, 0)),
            pl.BlockSpec((1, 1, EBLK), lambda g: (g, 0, 0)),
            pl.BlockSpec((1, 1, EBLK), lambda g: (g, 0, 0)),
            pl.BlockSpec((EBLK, EDIM), lambda g: (g, 0)),
            pl.BlockSpec((N, SDIM), lambda g: (0, 0)),
            pl.BlockSpec((EDIM, SDIM), lambda g: (0, 0)),
            pl.BlockSpec((1, SDIM), lambda g: (0, 0)),
            pl.BlockSpec((1, SDIM), lambda g: (0, 0)),
            pl.BlockSpec((SDIM, NBT), lambda g: (0, 0)),
            pl.BlockSpec((1, NBT), lambda g: (0, 0)),
        ],
        out_specs=pl.BlockSpec((EBLK, NBT), lambda g: (g, 0)),
        out_shape=jax.ShapeDtypeStruct((E, NBT), _F32),
    )(iR, jR, dR, es, t, Wc2, wd, bvec, W_b1, bb1)


def kernel(s, v, p, e, batch, edge_index_global,
           W_shared, b_shared, W_bond, b_bond,
           W_b0, b_b0, W_b1, b_b1,
           W_coords, W_atoms, b_atoms):
    try:
        info = plsc.get_sparse_core_info()
        nc, ns = info.num_cores, info.num_subcores
    except Exception:
        nc, ns = 2, 16

    j_ = edge_index_global[0].astype(jnp.int32)
    i_ = edge_index_global[1].astype(jnp.int32)
    batch2 = batch.astype(jnp.int32).reshape(1, N)
    vT = v.reshape(N, 3 * VDIM).T
    pT = p.T
    z = jnp.zeros((VDIM, 1), _F32)
    W3 = jnp.concatenate([
        jnp.concatenate([W_coords, z, z], axis=0),
        jnp.concatenate([z, W_coords, z], axis=0),
        jnp.concatenate([z, z, W_coords], axis=0),
    ], axis=1)
    W3T = W3.T
    W0 = W_b0[:SDIM]
    wd = W_b0[SDIM:SDIM + 1]
    bsh = b_shared.reshape(1, SDIM)
    bat = b_atoms.reshape(1, NAF)
    bb0 = b_b0.reshape(1, SDIM)
    bbond = b_bond.reshape(1, SDIM)
    bb1 = b_b1.reshape(1, NBT)
    epad = jnp.pad(e, ((0, PAD_ROWS), (0, 128 - EDIM)))

    t, atoms_pred, cpT, Wc2, bvec = _stage_a(
        s, vT, pT, batch2, W_shared, bsh, W3T, W_atoms, bat, W0, bb0,
        W_bond, bbond)

    keyf = j_ * N + i_
    keyr = i_ * N + j_
    m1 = jnp.full(((N * N) // (nc * ns),), -1, jnp.int32)
    T = _make_sc_table(nc, ns)(keyf, m1)
    cpflat = cpT.reshape(3 * N)
    es, dsq = _make_sc_gather(nc, ns)(T, j_, i_, keyf, keyr, epad, cpflat)

    iR = i_.reshape(NBLK, 1, EBLK)
    jR = j_.reshape(NBLK, 1, EBLK)
    dR = dsq.reshape(NBLK, 1, EBLK)
    es2 = es.reshape(E, EDIM)
    bonds_pred = _stage_c(iR, jR, dR, es2, t, Wc2, wd, bvec, W_b1, bb1)

    coords_pred = cpT.T
    return coords_pred, atoms_pred, bonds_pred


# speedup vs baseline: 31.9591x; 1.0955x over previous
import functools

import jax
import jax.numpy as jnp
from jax import lax
from jax.experimental import pallas as pl
from jax.experimental.pallas import tpu as pltpu
from jax.experimental.pallas import tpu_sc as plsc

N = 1024
E = 65536
SDIM = 256
VDIM = 64
EDIM = 16
NAF = 16
NBT = 5
NG = 64

PAD_ROWS = 256
EBLK = 1024
NBLK = E // EBLK
CH = 4096
SUB = 128

_F32 = jnp.float32
_HI = lax.Precision.HIGHEST


def _stage_a_body(s_ref, vT_ref, pT_ref, batch_ref, Wsh_ref, bsh_ref,
                  W3T_ref, Wat_ref, bat_ref, W0_ref, bb0_ref, Wbond_ref,
                  bbond_ref, t_ref, atoms_ref, cpT_ref, Wc2_ref, bvec_ref):
    s2 = jax.nn.silu(
        jnp.dot(s_ref[...], Wsh_ref[...], preferred_element_type=_F32,
                precision=_HI) + bsh_ref[...])
    t_ref[...] = jnp.dot(s2, W0_ref[...], preferred_element_type=_F32,
                         precision=_HI)
    atoms_ref[...] = jnp.dot(s2, Wat_ref[...], preferred_element_type=_F32,
                             precision=_HI) + bat_ref[...]
    cp0 = jnp.dot(W3T_ref[...], vT_ref[...], preferred_element_type=_F32,
                  precision=_HI) + pT_ref[...]
    bI = (lax.broadcasted_iota(jnp.int32, (NG, N), 0)
          == batch_ref[...]).astype(_F32)
    counts_row = lax.dot_general(jnp.ones((1, N), _F32), bI,
                                 (((1,), (1,)), ((), ())),
                                 preferred_element_type=_F32)
    segT = lax.dot_general(cp0, bI, (((1,), (1,)), ((), ())),
                           preferred_element_type=_F32, precision=_HI)
    meanT = segT / jnp.maximum(counts_row, 1.0)
    cpT_ref[...] = cp0 - jnp.dot(meanT, bI, preferred_element_type=_F32,
                                 precision=_HI)
    Wc2_ref[...] = 0.5 * jnp.dot(Wbond_ref[...], W0_ref[...],
                                 preferred_element_type=_F32, precision=_HI)
    bvec_ref[...] = jnp.dot(bbond_ref[...], W0_ref[...],
                            preferred_element_type=_F32,
                            precision=_HI) + bb0_ref[...]


def _stage_a(s, vT, pT, batch2, W_shared, bsh, W3T, W_atoms, bat, W0, bb0,
             W_bond, bbond):
    return pl.pallas_call(
        _stage_a_body,
        out_shape=[
            jax.ShapeDtypeStruct((N, SDIM), _F32),
            jax.ShapeDtypeStruct((N, NAF), _F32),
            jax.ShapeDtypeStruct((3, N), _F32),
            jax.ShapeDtypeStruct((EDIM, SDIM), _F32),
            jax.ShapeDtypeStruct((1, SDIM), _F32),
        ],
    )(s, vT, pT, batch2, W_shared, bsh, W3T, W_atoms, bat, W0, bb0, W_bond,
      bbond)


def _make_sc_table(nc, ns):
    nw = nc * ns
    spw = (N * N) // nw
    mesh = plsc.VectorSubcoreMesh(core_axis_name="c", subcore_axis_name="s",
                                  num_cores=nc, num_subcores=ns)

    @functools.partial(
        pl.kernel,
        out_type=jax.ShapeDtypeStruct((N * N,), jnp.int32),
        mesh=mesh,
        compiler_params=pltpu.CompilerParams(needs_layout_passes=False),
        scratch_types=[
            pltpu.VMEM((CH,), jnp.int32),
            pltpu.VMEM((CH,), jnp.int32),
            pltpu.VMEM((spw,), jnp.int32),
            pltpu.SemaphoreType.DMA,
            pltpu.SemaphoreType.DMA,
        ],
    )
    def sc_table(key_hbm, m1_hbm, T_hbm, kvA, kvB, Lb, semA, semB):
        wid = lax.axis_index("s") * nc + lax.axis_index("c")
        lo = wid * spw
        pltpu.sync_copy(m1_hbm, Lb)

        lanes = lax.iota(jnp.int32, 16)
        kvs = (kvA, kvB)
        sems = (semA, semB)
        nch = E // CH
        cps = [None, None]
        cps[0] = pltpu.async_copy(key_hbm.at[pl.ds(0, CH)], kvA, semA)
        for c in range(nch):
            if c + 1 < nch:
                cps[(c + 1) % 2] = pltpu.async_copy(
                    key_hbm.at[pl.ds((c + 1) * CH, CH)],
                    kvs[(c + 1) % 2], sems[(c + 1) % 2])
            cps[c % 2].wait()
            kv = kvs[c % 2]

            def scan(q, carry2):
                for u in range(4):
                    qq = q * 4 + u
                    k16 = kv[pl.ds(qq * 16, 16)]
                    loc = k16 - lo
                    m = (loc >= 0) & (loc < spw)
                    kvec = lanes + (c * CH + qq * 16)
                    plsc.store_scatter(Lb, [loc], kvec, mask=m)
                return carry2
            lax.fori_loop(0, CH // 64, scan, 0)
        pltpu.sync_copy(Lb, T_hbm.at[pl.ds(lo, spw)])

    return sc_table


def _make_sc_gather(nc, ns):
    nw = nc * ns
    epw = E // nw
    nsub = epw // SUB
    mesh = plsc.VectorSubcoreMesh(core_axis_name="c", subcore_axis_name="s",
                                  num_cores=nc, num_subcores=ns)

    @functools.partial(
        pl.kernel,
        out_type=[
            jax.ShapeDtypeStruct((E * EDIM,), _F32),
            jax.ShapeDtypeStruct((E,), _F32),
        ],
        mesh=mesh,
        compiler_params=pltpu.CompilerParams(needs_layout_passes=False),
        scratch_types=[
            pltpu.VMEM((epw,), jnp.int32),
            pltpu.VMEM((epw,), jnp.int32),
            pltpu.VMEM((epw,), jnp.int32),
            pltpu.VMEM((epw,), jnp.int32),
            pltpu.VMEM((epw,), jnp.int32),
            pltpu.VMEM((epw,), jnp.int32),
            pltpu.VMEM((epw * EDIM,), _F32),
            pltpu.VMEM((SUB, 128), _F32),
            pltpu.VMEM((SUB, 128), _F32),
            pltpu.VMEM((epw,), _F32),
            pltpu.VMEM((N,), _F32),
            pltpu.VMEM((N,), _F32),
            pltpu.VMEM((N,), _F32),
            pltpu.SemaphoreType.DMA,
            pltpu.SemaphoreType.DMA,
        ],
    )
    def sc_gather(T_hbm, j_hbm, i_hbm, keyf_hbm, keyr_hbm, epad_hbm, cpT_hbm,
                  es_hbm, d_hbm,
                  jv, iv, keyb, rkeyb, fidb, ridb, efb,
                  slabA, slabB, db, cx, cy, cz, semA, semB):
        wid = lax.axis_index("s") * nc + lax.axis_index("c")
        base = wid * epw
        pltpu.sync_copy(j_hbm.at[pl.ds(base, epw)], jv)
        pltpu.sync_copy(i_hbm.at[pl.ds(base, epw)], iv)
        pltpu.sync_copy(keyf_hbm.at[pl.ds(base, epw)], keyb)
        pltpu.sync_copy(keyr_hbm.at[pl.ds(base, epw)], rkeyb)
        pltpu.sync_copy(cpT_hbm.at[pl.ds(0, N)], cx)
        pltpu.sync_copy(cpT_hbm.at[pl.ds(N, N)], cy)
        pltpu.sync_copy(cpT_hbm.at[pl.ds(2 * N, N)], cz)

        cpsT = []
        for b in range(nsub):
            sl = pl.ds(b * SUB, SUB)
            cpsT.append(pltpu.async_copy(T_hbm.at[keyb.at[sl]], fidb.at[sl],
                                         semA))
        for b in range(nsub):
            sl = pl.ds(b * SUB, SUB)
            cpsT.append(pltpu.async_copy(T_hbm.at[rkeyb.at[sl]], ridb.at[sl],
                                         semA))

        def phase1(q, carry):
            sl = pl.ds(q * 16, 16)
            j16 = jv[sl]
            i16 = iv[sl]
            xi = plsc.load_gather(cx, [i16])
            xj = plsc.load_gather(cx, [j16])
            yi = plsc.load_gather(cy, [i16])
            yj = plsc.load_gather(cy, [j16])
            zi = plsc.load_gather(cz, [i16])
            zj = plsc.load_gather(cz, [j16])
            dx = xi - xj
            dy = yi - yj
            dz = zi - zj
            db[sl] = dx * dx + dy * dy + dz * dz
            return carry
        lax.fori_loop(0, epw // 16, phase1, 0)
        pltpu.sync_copy(db, d_hbm.at[pl.ds(base, epw)])

        for cp in cpsT:
            cp.wait()

        def phase2(q, carry):
            sl = pl.ds(q * 16, 16)
            r16 = ridb[sl]
            pad = E + ((lax.iota(jnp.int32, 16) + (q * 16 + base))
                       & (PAD_ROWS - 1))
            ridb[sl] = jnp.where(r16 < 0, pad, r16)
            return carry
        lax.fori_loop(0, epw // 16, phase2, 0)

        def run_dir(idx_ref, out_ref, accumulate):
            slabs = (slabA, slabB)
            sems = (semA, semB)
            cps = [None, None]
            cps[0] = pltpu.async_copy(epad_hbm.at[idx_ref.at[pl.ds(0, SUB)]],
                                      slabs[0], sems[0])
            for b in range(nsub):
                if b + 1 < nsub:
                    cps[(b + 1) % 2] = pltpu.async_copy(
                        epad_hbm.at[idx_ref.at[pl.ds((b + 1) * SUB, SUB)]],
                        slabs[(b + 1) % 2], sems[(b + 1) % 2])
                cps[b % 2].wait()
                slab = slabs[b % 2]

                def extract(r, carry):
                    sl16 = pl.ds((b * SUB + r) * 16, 16)
                    row = slab[r, pl.ds(0, 16)]
                    if accumulate:
                        row = row + out_ref[sl16]
                    out_ref[sl16] = row
                    return carry
                lax.fori_loop(0, SUB, extract, 0)

        run_dir(fidb, efb, False)
        run_dir(ridb, efb, True)

        pltpu.sync_copy(efb, es_hbm.at[pl.ds(base * EDIM, epw * EDIM)])

    return sc_gather


def _stage_c_body(iR, jR, dR, esR, tR, Wc2R, wdR, bvecR, Wb1R, bb1R, oR):
    irow = iR[0]
    jrow = jR[0]
    nn = lax.broadcasted_iota(jnp.int32, (N, EBLK), 0)
    ohT = (nn == irow).astype(_F32) + (nn == jrow).astype(_F32)
    tn = (((0,), (0,)), ((), ()))
    h = lax.dot_general(ohT, tR[...], tn, preferred_element_type=_F32)
    h = h + jnp.dot(esR[...], Wc2R[...], preferred_element_type=_F32)
    h = h + lax.dot_general(dR[0], wdR[...], tn,
                            preferred_element_type=_F32)
    h = h + bvecR[...]
    sil = h * jax.nn.sigmoid(h)
    oR[...] = jnp.dot(sil, Wb1R[...], preferred_element_type=_F32) + bb1R[...]


def _stage_c(iR, jR, dR, es, t, Wc2, wd, bvec, W_b1, bb1):
    return pl.pallas_call(
        _stage_c_body,
        grid=(NBLK,),
        in_specs=[
            pl.BlockSpec((1, 1, EBLK), lambda g: (g, 0, 0)),
            pl.BlockSpec((1, 1, EBLK), lambda g: (g, 0, 0)),
            pl.BlockSpec((1, 1, EBLK), lambda g: (g, 0, 0)),
            pl.BlockSpec((EBLK, EDIM), lambda g: (g, 0)),
            pl.BlockSpec((N, SDIM), lambda g: (0, 0)),
            pl.BlockSpec((EDIM, SDIM), lambda g: (0, 0)),
            pl.BlockSpec((1, SDIM), lambda g: (0, 0)),
            pl.BlockSpec((1, SDIM), lambda g: (0, 0)),
            pl.BlockSpec((SDIM, NBT), lambda g: (0, 0)),
            pl.BlockSpec((1, NBT), lambda g: (0, 0)),
        ],
        out_specs=pl.BlockSpec((EBLK, NBT), lambda g: (g, 0)),
        out_shape=jax.ShapeDtypeStruct((E, NBT), _F32),
    )(iR, jR, dR, es, t, Wc2, wd, bvec, W_b1, bb1)


def kernel(s, v, p, e, batch, edge_index_global,
           W_shared, b_shared, W_bond, b_bond,
           W_b0, b_b0, W_b1, b_b1,
           W_coords, W_atoms, b_atoms):
    try:
        info = plsc.get_sparse_core_info()
        nc, ns = info.num_cores, info.num_subcores
    except Exception:
        nc, ns = 2, 16

    j_ = edge_index_global[0].astype(jnp.int32)
    i_ = edge_index_global[1].astype(jnp.int32)
    batch2 = batch.astype(jnp.int32).reshape(1, N)
    vT = v.reshape(N, 3 * VDIM).T
    pT = p.T
    z = jnp.zeros((VDIM, 1), _F32)
    W3 = jnp.concatenate([
        jnp.concatenate([W_coords, z, z], axis=0),
        jnp.concatenate([z, W_coords, z], axis=0),
        jnp.concatenate([z, z, W_coords], axis=0),
    ], axis=1)
    W3T = W3.T
    W0 = W_b0[:SDIM]
    wd = W_b0[SDIM:SDIM + 1]
    bsh = b_shared.reshape(1, SDIM)
    bat = b_atoms.reshape(1, NAF)
    bb0 = b_b0.reshape(1, SDIM)
    bbond = b_bond.reshape(1, SDIM)
    bb1 = b_b1.reshape(1, NBT)
    epad = jnp.pad(e, ((0, PAD_ROWS), (0, 128 - EDIM)))

    t, atoms_pred, cpT, Wc2, bvec = _stage_a(
        s, vT, pT, batch2, W_shared, bsh, W3T, W_atoms, bat, W0, bb0,
        W_bond, bbond)

    keyf = j_ * N + i_
    keyr = i_ * N + j_
    m1 = jnp.full(((N * N) // (nc * ns),), -1, jnp.int32)
    T = _make_sc_table(nc, ns)(keyf, m1)
    cpflat = cpT.reshape(3 * N)
    es, dsq = _make_sc_gather(nc, ns)(T, j_, i_, keyf, keyr, epad, cpflat)

    iR = i_.reshape(NBLK, 1, EBLK)
    jR = j_.reshape(NBLK, 1, EBLK)
    dR = dsq.reshape(NBLK, 1, EBLK)
    es2 = es.reshape(E, EDIM)
    bonds_pred = _stage_c(iR, jR, dR, es2, t, Wc2, wd, bvec, W_b1, bb1)

    coords_pred = cpT.T
    return coords_pred, atoms_pred, bonds_pred


# speedup vs baseline: 32.3828x; 1.0133x over previous
import functools

import jax
import jax.numpy as jnp
from jax import lax
from jax.experimental import pallas as pl
from jax.experimental.pallas import tpu as pltpu
from jax.experimental.pallas import tpu_sc as plsc

N = 1024
E = 65536
SDIM = 256
VDIM = 64
EDIM = 16
NAF = 16
NBT = 5
NG = 64

PAD_ROWS = 256
EBLK = 2048
NBLK = E // EBLK
CH = 4096
SUB = 128

_F32 = jnp.float32
_HI = lax.Precision.HIGHEST


def _stage_a_body(s_ref, vT_ref, pT_ref, batch_ref, Wsh_ref, bsh_ref,
                  W3T_ref, Wat_ref, bat_ref, W0_ref, bb0_ref, Wbond_ref,
                  bbond_ref, t_ref, atoms_ref, cpT_ref, Wc2_ref, bvec_ref):
    s2 = jax.nn.silu(
        jnp.dot(s_ref[...], Wsh_ref[...], preferred_element_type=_F32,
                precision=_HI) + bsh_ref[...])
    t_ref[...] = jnp.dot(s2, W0_ref[...], preferred_element_type=_F32,
                         precision=_HI)
    atoms_ref[...] = jnp.dot(s2, Wat_ref[...], preferred_element_type=_F32,
                             precision=_HI) + bat_ref[...]
    cp0 = jnp.dot(W3T_ref[...], vT_ref[...], preferred_element_type=_F32,
                  precision=_HI) + pT_ref[...]
    bI = (lax.broadcasted_iota(jnp.int32, (NG, N), 0)
          == batch_ref[...]).astype(_F32)
    counts_row = lax.dot_general(jnp.ones((1, N), _F32), bI,
                                 (((1,), (1,)), ((), ())),
                                 preferred_element_type=_F32)
    segT = lax.dot_general(cp0, bI, (((1,), (1,)), ((), ())),
                           preferred_element_type=_F32, precision=_HI)
    meanT = segT / jnp.maximum(counts_row, 1.0)
    cpT_ref[...] = cp0 - jnp.dot(meanT, bI, preferred_element_type=_F32,
                                 precision=_HI)
    Wc2_ref[...] = 0.5 * jnp.dot(Wbond_ref[...], W0_ref[...],
                                 preferred_element_type=_F32, precision=_HI)
    bvec_ref[...] = jnp.dot(bbond_ref[...], W0_ref[...],
                            preferred_element_type=_F32,
                            precision=_HI) + bb0_ref[...]


def _stage_a(s, vT, pT, batch2, W_shared, bsh, W3T, W_atoms, bat, W0, bb0,
             W_bond, bbond):
    return pl.pallas_call(
        _stage_a_body,
        out_shape=[
            jax.ShapeDtypeStruct((N, SDIM), _F32),
            jax.ShapeDtypeStruct((N, NAF), _F32),
            jax.ShapeDtypeStruct((3, N), _F32),
            jax.ShapeDtypeStruct((EDIM, SDIM), _F32),
            jax.ShapeDtypeStruct((1, SDIM), _F32),
        ],
    )(s, vT, pT, batch2, W_shared, bsh, W3T, W_atoms, bat, W0, bb0, W_bond,
      bbond)


def _make_sc_table(nc, ns):
    nw = nc * ns
    spw = (N * N) // nw
    mesh = plsc.VectorSubcoreMesh(core_axis_name="c", subcore_axis_name="s",
                                  num_cores=nc, num_subcores=ns)

    @functools.partial(
        pl.kernel,
        out_type=jax.ShapeDtypeStruct((N * N,), jnp.int32),
        mesh=mesh,
        compiler_params=pltpu.CompilerParams(needs_layout_passes=False),
        scratch_types=[
            pltpu.VMEM((CH,), jnp.int32),
            pltpu.VMEM((CH,), jnp.int32),
            pltpu.VMEM((spw,), jnp.int32),
            pltpu.SemaphoreType.DMA,
            pltpu.SemaphoreType.DMA,
        ],
    )
    def sc_table(key_hbm, m1_hbm, T_hbm, kvA, kvB, Lb, semA, semB):
        wid = lax.axis_index("s") * nc + lax.axis_index("c")
        lo = wid * spw
        pltpu.sync_copy(m1_hbm, Lb)

        lanes = lax.iota(jnp.int32, 16)
        kvs = (kvA, kvB)
        sems = (semA, semB)
        nch = E // CH
        cps = [None, None]
        cps[0] = pltpu.async_copy(key_hbm.at[pl.ds(0, CH)], kvA, semA)
        for c in range(nch):
            if c + 1 < nch:
                cps[(c + 1) % 2] = pltpu.async_copy(
                    key_hbm.at[pl.ds((c + 1) * CH, CH)],
                    kvs[(c + 1) % 2], sems[(c + 1) % 2])
            cps[c % 2].wait()
            kv = kvs[c % 2]

            def scan(q, carry2):
                for u in range(4):
                    qq = q * 4 + u
                    k16 = kv[pl.ds(qq * 16, 16)]
                    loc = k16 - lo
                    m = (loc >= 0) & (loc < spw)
                    kvec = lanes + (c * CH + qq * 16)
                    plsc.store_scatter(Lb, [loc], kvec, mask=m)
                return carry2
            lax.fori_loop(0, CH // 64, scan, 0)
        pltpu.sync_copy(Lb, T_hbm.at[pl.ds(lo, spw)])

    return sc_table


def _make_sc_gather(nc, ns):
    nw = nc * ns
    epw = E // nw
    nsub = epw // SUB
    mesh = plsc.VectorSubcoreMesh(core_axis_name="c", subcore_axis_name="s",
                                  num_cores=nc, num_subcores=ns)

    @functools.partial(
        pl.kernel,
        out_type=[
            jax.ShapeDtypeStruct((E * EDIM,), _F32),
            jax.ShapeDtypeStruct((E,), _F32),
        ],
        mesh=mesh,
        compiler_params=pltpu.CompilerParams(needs_layout_passes=False),
        scratch_types=[
            pltpu.VMEM((epw,), jnp.int32),
            pltpu.VMEM((epw,), jnp.int32),
            pltpu.VMEM((epw,), jnp.int32),
            pltpu.VMEM((epw,), jnp.int32),
            pltpu.VMEM((epw,), jnp.int32),
            pltpu.VMEM((epw,), jnp.int32),
            pltpu.VMEM((epw * EDIM,), _F32),
            pltpu.VMEM((SUB, 128), _F32),
            pltpu.VMEM((SUB, 128), _F32),
            pltpu.VMEM((epw,), _F32),
            pltpu.VMEM((N,), _F32),
            pltpu.VMEM((N,), _F32),
            pltpu.VMEM((N,), _F32),
            pltpu.SemaphoreType.DMA,
            pltpu.SemaphoreType.DMA,
        ],
    )
    def sc_gather(T_hbm, j_hbm, i_hbm, keyf_hbm, keyr_hbm, epad_hbm, cpT_hbm,
                  es_hbm, d_hbm,
                  jv, iv, keyb, rkeyb, fidb, ridb, efb,
                  slabA, slabB, db, cx, cy, cz, semA, semB):
        wid = lax.axis_index("s") * nc + lax.axis_index("c")
        base = wid * epw
        pltpu.sync_copy(j_hbm.at[pl.ds(base, epw)], jv)
        pltpu.sync_copy(i_hbm.at[pl.ds(base, epw)], iv)
        pltpu.sync_copy(keyf_hbm.at[pl.ds(base, epw)], keyb)
        pltpu.sync_copy(keyr_hbm.at[pl.ds(base, epw)], rkeyb)
        pltpu.sync_copy(cpT_hbm.at[pl.ds(0, N)], cx)
        pltpu.sync_copy(cpT_hbm.at[pl.ds(N, N)], cy)
        pltpu.sync_copy(cpT_hbm.at[pl.ds(2 * N, N)], cz)

        cpsT = []
        for b in range(nsub):
            sl = pl.ds(b * SUB, SUB)
            cpsT.append(pltpu.async_copy(T_hbm.at[keyb.at[sl]], fidb.at[sl],
                                         semA))
        for b in range(nsub):
            sl = pl.ds(b * SUB, SUB)
            cpsT.append(pltpu.async_copy(T_hbm.at[rkeyb.at[sl]], ridb.at[sl],
                                         semA))

        def phase1(q, carry):
            sl = pl.ds(q * 16, 16)
            j16 = jv[sl]
            i16 = iv[sl]
            xi = plsc.load_gather(cx, [i16])
            xj = plsc.load_gather(cx, [j16])
            yi = plsc.load_gather(cy, [i16])
            yj = plsc.load_gather(cy, [j16])
            zi = plsc.load_gather(cz, [i16])
            zj = plsc.load_gather(cz, [j16])
            dx = xi - xj
            dy = yi - yj
            dz = zi - zj
            db[sl] = dx * dx + dy * dy + dz * dz
            return carry
        lax.fori_loop(0, epw // 16, phase1, 0)
        pltpu.sync_copy(db, d_hbm.at[pl.ds(base, epw)])

        for cp in cpsT:
            cp.wait()

        def phase2(q, carry):
            sl = pl.ds(q * 16, 16)
            r16 = ridb[sl]
            pad = E + ((lax.iota(jnp.int32, 16) + (q * 16 + base))
                       & (PAD_ROWS - 1))
            ridb[sl] = jnp.where(r16 < 0, pad, r16)
            return carry
        lax.fori_loop(0, epw // 16, phase2, 0)

        def run_dir(idx_ref, out_ref, accumulate):
            slabs = (slabA, slabB)
            sems = (semA, semB)
            cps = [None, None]
            cps[0] = pltpu.async_copy(epad_hbm.at[idx_ref.at[pl.ds(0, SUB)]],
                                      slabs[0], sems[0])
            for b in range(nsub):
                if b + 1 < nsub:
                    cps[(b + 1) % 2] = pltpu.async_copy(
                        epad_hbm.at[idx_ref.at[pl.ds((b + 1) * SUB, SUB)]],
                        slabs[(b + 1) % 2], sems[(b + 1) % 2])
                cps[b % 2].wait()
                slab = slabs[b % 2]

                def extract(r, carry):
                    sl16 = pl.ds((b * SUB + r) * 16, 16)
                    row = slab[r, pl.ds(0, 16)]
                    if accumulate:
                        row = row + out_ref[sl16]
                    out_ref[sl16] = row
                    return carry
                lax.fori_loop(0, SUB, extract, 0)

        run_dir(fidb, efb, False)
        run_dir(ridb, efb, True)

        pltpu.sync_copy(efb, es_hbm.at[pl.ds(base * EDIM, epw * EDIM)])

    return sc_gather


def _stage_c_body(iR, jR, dR, esR, tR, Wc2R, wdR, bvecR, Wb1R, bb1R, oR):
    irow = iR[0]
    jrow = jR[0]
    nn = lax.broadcasted_iota(jnp.int32, (N, EBLK), 0)
    ohT = (nn == irow).astype(_F32) + (nn == jrow).astype(_F32)
    tn = (((0,), (0,)), ((), ()))
    h = lax.dot_general(ohT, tR[...], tn, preferred_element_type=_F32)
    h = h + jnp.dot(esR[...], Wc2R[...], preferred_element_type=_F32)
    h = h + lax.dot_general(dR[0], wdR[...], tn,
                            preferred_element_type=_F32)
    h = h + bvecR[...]
    sil = h * jax.nn.sigmoid(h)
    oR[...] = jnp.dot(sil, Wb1R[...], preferred_element_type=_F32) + bb1R[...]


def _stage_c(iR, jR, dR, es, t, Wc2, wd, bvec, W_b1, bb1):
    return pl.pallas_call(
        _stage_c_body,
        grid=(NBLK,),
        in_specs=[
            pl.BlockSpec((1, 1, EBLK), lambda g: (g, 0, 0)),
            pl.BlockSpec((1, 1, EBLK), lambda g: (g, 0, 0)),
            pl.BlockSpec((1, 1, EBLK), lambda g: (g, 0, 0)),
            pl.BlockSpec((EBLK, EDIM), lambda g: (g, 0)),
            pl.BlockSpec((N, SDIM), lambda g: (0, 0)),
            pl.BlockSpec((EDIM, SDIM), lambda g: (0, 0)),
            pl.BlockSpec((1, SDIM), lambda g: (0, 0)),
            pl.BlockSpec((1, SDIM), lambda g: (0, 0)),
            pl.BlockSpec((SDIM, NBT), lambda g: (0, 0)),
            pl.BlockSpec((1, NBT), lambda g: (0, 0)),
        ],
        out_specs=pl.BlockSpec((EBLK, NBT), lambda g: (g, 0)),
        out_shape=jax.ShapeDtypeStruct((E, NBT), _F32),
    )(iR, jR, dR, es, t, Wc2, wd, bvec, W_b1, bb1)


def kernel(s, v, p, e, batch, edge_index_global,
           W_shared, b_shared, W_bond, b_bond,
           W_b0, b_b0, W_b1, b_b1,
           W_coords, W_atoms, b_atoms):
    try:
        info = plsc.get_sparse_core_info()
        nc, ns = info.num_cores, info.num_subcores
    except Exception:
        nc, ns = 2, 16

    j_ = edge_index_global[0].astype(jnp.int32)
    i_ = edge_index_global[1].astype(jnp.int32)
    batch2 = batch.astype(jnp.int32).reshape(1, N)
    vT = v.reshape(N, 3 * VDIM).T
    pT = p.T
    z = jnp.zeros((VDIM, 1), _F32)
    W3 = jnp.concatenate([
        jnp.concatenate([W_coords, z, z], axis=0),
        jnp.concatenate([z, W_coords, z], axis=0),
        jnp.concatenate([z, z, W_coords], axis=0),
    ], axis=1)
    W3T = W3.T
    W0 = W_b0[:SDIM]
    wd = W_b0[SDIM:SDIM + 1]
    bsh = b_shared.reshape(1, SDIM)
    bat = b_atoms.reshape(1, NAF)
    bb0 = b_b0.reshape(1, SDIM)
    bbond = b_bond.reshape(1, SDIM)
    bb1 = b_b1.reshape(1, NBT)
    epad = jnp.pad(e, ((0, PAD_ROWS), (0, 128 - EDIM)))

    t, atoms_pred, cpT, Wc2, bvec = _stage_a(
        s, vT, pT, batch2, W_shared, bsh, W3T, W_atoms, bat, W0, bb0,
        W_bond, bbond)

    keyf = j_ * N + i_
    keyr = i_ * N + j_
    m1 = jnp.full(((N * N) // (nc * ns),), -1, jnp.int32)
    T = _make_sc_table(nc, ns)(keyf, m1)
    cpflat = cpT.reshape(3 * N)
    es, dsq = _make_sc_gather(nc, ns)(T, j_, i_, keyf, keyr, epad, cpflat)

    iR = i_.reshape(NBLK, 1, EBLK)
    jR = j_.reshape(NBLK, 1, EBLK)
    dR = dsq.reshape(NBLK, 1, EBLK)
    es2 = es.reshape(E, EDIM)
    bonds_pred = _stage_c(iR, jR, dR, es2, t, Wc2, wd, bvec, W_b1, bb1)

    coords_pred = cpT.T
    return coords_pred, atoms_pred, bonds_pred
